# Initial kernel scaffold; baseline (speedup 1.0000x reference)
import functools

import jax
import jax.numpy as jnp
from jax import lax
from jax.experimental import pallas as pl
from jax.experimental.pallas import tpu as pltpu
from jax.experimental.pallas import tpu_sc as plsc

N = 10000
E = 320000
D = 128
P = 64
H = 128
R = 101
G = 16

NC = 2
NS = 16
NW = NC * NS
EW = E // NW
K = 80
CHUNKS = EW // K
RT = N // NS
ZR = 125
F32 = jnp.float32
EPS = 1e-15
PREC = lax.Precision.HIGHEST

_MESH = plsc.VectorSubcoreMesh(core_axis_name="c", subcore_axis_name="s")


def _dot(a, b):
    return jnp.dot(a, b, preferred_element_type=F32, precision=PREC)


def _zero_acc_rows(zbuf, acc, s):
    zv = jnp.zeros((16,), F32)

    def zrow(i, carry):
        for j in range(D // 16):
            zbuf[i, pl.ds(j * 16, 16)] = zv
        return carry

    lax.fori_loop(0, ZR, zrow, 0)
    for k in range(RT // ZR):
        pltpu.sync_copy(zbuf, acc.at[pl.ds(s * RT + k * ZR, ZR)])


def _copy_out_rows(zbuf, acc, out_hbm, c, s):
    for k in range(RT // ZR):
        sl = pl.ds(s * RT + k * ZR, ZR)
        pltpu.sync_copy(acc.at[sl], zbuf)
        pltpu.sync_copy(zbuf, out_hbm.at[c, sl])


def _sc_seg_rows(h, src, dst, mask=None):
    has_mask = mask is not None

    def body(*refs):
        if has_mask:
            (h_hbm, src_hbm, dst_hbm, m_hbm, out_hbm,
             sbuf, dbuf, mbuf, rows, zbuf, acc, sem) = refs
        else:
            (h_hbm, src_hbm, dst_hbm, out_hbm,
             sbuf, dbuf, rows, zbuf, acc, sem) = refs
        c = lax.axis_index("c")
        s = lax.axis_index("s")
        wid = c * NS + s
        _zero_acc_rows(zbuf, acc, s)
        plsc.subcore_barrier()

        def chunk(ci, carry):
            base = wid * EW + ci * K
            pltpu.sync_copy(src_hbm.at[pl.ds(base, K)], sbuf)
            pltpu.sync_copy(dst_hbm.at[pl.ds(base, K)], dbuf)
            pltpu.async_copy(h_hbm.at[sbuf], rows, sem).wait()
            if has_mask:
                pltpu.sync_copy(m_hbm.at[pl.ds(base, K)], mbuf)

                def mrow(i, cc):
                    m = mbuf[i]
                    for j in range(D // 16):
                        sl = pl.ds(j * 16, 16)
                        rows[i, sl] = rows[i, sl] * m
                    return cc

                lax.fori_loop(0, K, mrow, 0)
            pltpu.sync_copy(rows, acc.at[dbuf], add=True)
            return carry

        lax.fori_loop(0, CHUNKS, chunk, 0)
        plsc.subcore_barrier()
        _copy_out_rows(zbuf, acc, out_hbm, c, s)

    scratch = [
        pltpu.VMEM((K,), jnp.int32),
        pltpu.VMEM((K,), jnp.int32),
    ]
    if has_mask:
        scratch.append(pltpu.VMEM((K,), F32))
    scratch += [
        pltpu.VMEM((K, D), F32),
        pltpu.VMEM((ZR, D), F32),
        pltpu.VMEM_SHARED((N, D), F32),
        pltpu.SemaphoreType.DMA,
    ]
    fn = pl.kernel(
        body,
        out_type=jax.ShapeDtypeStruct((NC, N, D), F32),
        mesh=_MESH,
        scratch_types=scratch,
    )
    args = (h, src, dst) + ((mask,) if has_mask else ())
    return fn(*args)


def _sc_counts(src, dst, etype, batch):
    def body(src_hbm, dst_hbm, t_hbm, b_hbm, outR_hbm, outG_hbm,
             sbuf, dbuf, tbuf, fR, fG, ones, bv, zb, accR, accG):
        c = lax.axis_index("c")
        s = lax.axis_index("s")
        wid = c * NS + s
        zv = jnp.zeros((16,), F32)
        ov = jnp.ones((16,), F32)

        def zfill(i, carry):
            zb[pl.ds(i * 16, 16)] = zv
            return carry

        lax.fori_loop(0, 10000 // 16, zfill, 0)
        for j in range(K // 16):
            ones[pl.ds(j * 16, 16)] = ov
        for k in range(8):
            pltpu.sync_copy(zb, accR.at[pl.ds(s * 80000 + k * 10000, 10000)])
        pltpu.sync_copy(zb, accG.at[pl.ds(s * 10000, 10000)])
        pltpu.sync_copy(b_hbm, bv)
        plsc.subcore_barrier()

        def chunk(ci, carry):
            base = wid * EW + ci * K
            pltpu.sync_copy(src_hbm.at[pl.ds(base, K)], sbuf)
            pltpu.sync_copy(dst_hbm.at[pl.ds(base, K)], dbuf)
            pltpu.sync_copy(t_hbm.at[pl.ds(base, K)], tbuf)
            for g in range(K // 16):
                sl = pl.ds(g * 16, 16)
                d16 = dbuf[sl]
                b16 = plsc.load_gather(bv, [sbuf[sl]])
                fR[sl] = d16 * 128 + tbuf[sl]
                fG[sl] = d16 * G + b16
            pltpu.sync_copy(ones, accR.at[fR], add=True)
            pltpu.sync_copy(ones, accG.at[fG], add=True)
            return carry

        lax.fori_loop(0, CHUNKS, chunk, 0)
        plsc.subcore_barrier()
        for k in range(8):
            sl = pl.ds(s * 80000 + k * 10000, 10000)
            pltpu.sync_copy(accR.at[sl], zb)
            pltpu.sync_copy(zb, outR_hbm.at[c, sl])
        sl = pl.ds(s * 10000, 10000)
        pltpu.sync_copy(accG.at[sl], zb)
        pltpu.sync_copy(zb, outG_hbm.at[c, sl])

    fn = pl.kernel(
        body,
        out_type=(
            jax.ShapeDtypeStruct((NC, N * 128), F32),
            jax.ShapeDtypeStruct((NC, N * G), F32),
        ),
        mesh=_MESH,
        scratch_types=[
            pltpu.VMEM((K,), jnp.int32),
            pltpu.VMEM((K,), jnp.int32),
            pltpu.VMEM((K,), jnp.int32),
            pltpu.VMEM((K,), jnp.int32),
            pltpu.VMEM((K,), jnp.int32),
            pltpu.VMEM((K,), F32),
            pltpu.VMEM((N,), jnp.int32),
            pltpu.VMEM((10000,), F32),
            pltpu.VMEM_SHARED((N * 128,), F32),
            pltpu.VMEM_SHARED((N * G,), F32),
        ],
    )
    outR, outG = fn(src, dst, etype, batch)
    return outR.reshape(NC, N, 128), outG.reshape(NC, N, G)


def _sc_wcounts(dst, etype, mask):
    def body(dst_hbm, t_hbm, m_hbm, outR_hbm,
             dbuf, tbuf, fR, wbuf, zb, accR):
        c = lax.axis_index("c")
        s = lax.axis_index("s")
        wid = c * NS + s
        zv = jnp.zeros((16,), F32)

        def zfill(i, carry):
            zb[pl.ds(i * 16, 16)] = zv
            return carry

        lax.fori_loop(0, 10000 // 16, zfill, 0)
        for k in range(8):
            pltpu.sync_copy(zb, accR.at[pl.ds(s * 80000 + k * 10000, 10000)])
        plsc.subcore_barrier()

        def chunk(ci, carry):
            base = wid * EW + ci * K
            pltpu.sync_copy(dst_hbm.at[pl.ds(base, K)], dbuf)
            pltpu.sync_copy(t_hbm.at[pl.ds(base, K)], tbuf)
            pltpu.sync_copy(m_hbm.at[pl.ds(base, K)], wbuf)
            for g in range(K // 16):
                sl = pl.ds(g * 16, 16)
                fR[sl] = dbuf[sl] * 128 + tbuf[sl]
            pltpu.sync_copy(wbuf, accR.at[fR], add=True)
            return carry

        lax.fori_loop(0, CHUNKS, chunk, 0)
        plsc.subcore_barrier()
        for k in range(8):
            sl = pl.ds(s * 80000 + k * 10000, 10000)
            pltpu.sync_copy(accR.at[sl], zb)
            pltpu.sync_copy(zb, outR_hbm.at[c, sl])

    fn = pl.kernel(
        body,
        out_type=jax.ShapeDtypeStruct((NC, N * 128), F32),
        mesh=_MESH,
        scratch_types=[
            pltpu.VMEM((K,), jnp.int32),
            pltpu.VMEM((K,), jnp.int32),
            pltpu.VMEM((K,), jnp.int32),
            pltpu.VMEM((K,), F32),
            pltpu.VMEM((10000,), F32),
            pltpu.VMEM_SHARED((N * 128,), F32),
        ],
    )
    return fn(dst, etype, mask).reshape(NC, N, 128)


def _sc_edge_feat(ha, hb, src, dst):
    def body(ha_hbm, hb_hbm, src_hbm, dst_hbm, out_hbm,
             sbuf, dbuf, rows_a, rows_b, sema, semb):
        c = lax.axis_index("c")
        s = lax.axis_index("s")
        wid = c * NS + s

        def chunk(ci, carry):
            base = wid * EW + ci * K
            pltpu.sync_copy(src_hbm.at[pl.ds(base, K)], sbuf)
            pltpu.sync_copy(dst_hbm.at[pl.ds(base, K)], dbuf)
            da = pltpu.async_copy(ha_hbm.at[sbuf], rows_a, sema)
            db = pltpu.async_copy(hb_hbm.at[dbuf], rows_b, semb)
            da.wait()
            db.wait()

            def arow(i, cc):
                for j in range(H // 16):
                    sl = pl.ds(j * 16, 16)
                    rows_a[i, sl] = rows_a[i, sl] + rows_b[i, sl]
                return cc

            lax.fori_loop(0, K, arow, 0)
            pltpu.sync_copy(rows_a, out_hbm.at[pl.ds(base, K)])
            return carry

        lax.fori_loop(0, CHUNKS, chunk, 0)

    fn = pl.kernel(
        body,
        out_type=jax.ShapeDtypeStruct((E, H), F32),
        mesh=_MESH,
        scratch_types=[
            pltpu.VMEM((K,), jnp.int32),
            pltpu.VMEM((K,), jnp.int32),
            pltpu.VMEM((K, H), F32),
            pltpu.VMEM((K, H), F32),
            pltpu.SemaphoreType.DMA,
            pltpu.SemaphoreType.DMA,
        ],
    )
    return fn(ha, hb, src, dst)


def _tc_prep1(cntR, cntG, eep, proto, w1a, w1b):
    def body(cr, cg, ee, pr, wa, wb, dinv, u1):
        crs = cr[0] + cr[1]
        cgs = cg[0] + cg[1]
        deg = jnp.maximum(jnp.sum(crs, axis=1, keepdims=True), 1.0)
        dinv[...] = 1.0 / deg
        u1[...] = _dot(_dot(crs, ee[...]), wa[...]) + \
            _dot(_dot(cgs, pr[...]), wb[...])

    return pl.pallas_call(
        body,
        out_shape=(
            jax.ShapeDtypeStruct((N, 1), F32),
            jax.ShapeDtypeStruct((N, H), F32),
        ),
    )(cntR, cntG, eep, proto, w1a, w1b)


def _tc_layer1(a_p, u1, x, w1a, ws1, dinv):
    def body(a, u, xr, wa, ws, dv, out):
        asum = a[0] + a[1]
        acc = (_dot(asum, wa[...]) + u[...]) * dv[...]
        out[...] = jnp.maximum(acc + _dot(xr[...], ws[...]), 0.0)

    return pl.pallas_call(
        body, out_shape=jax.ShapeDtypeStruct((N, H), F32),
    )(a_p, u1, x, w1a, ws1, dinv)


def _tc_layer(a_p, h, w, ws, dinv):
    def body(a, hr, wr, ws_r, dv, out):
        asum = a[0] + a[1]
        acc = _dot(asum, wr[...]) * dv[...]
        out[...] = jnp.maximum(acc + _dot(hr[...], ws_r[...]), 0.0)

    return pl.pallas_call(
        body, out_shape=jax.ShapeDtypeStruct((N, H), F32),
    )(a_p, h, w, ws, dinv)


def _tc_ab(h3, wea, web):
    def body(hr, wa, wb, oa, ob):
        oa[...] = _dot(hr[...], wa[...])
        ob[...] = _dot(hr[...], wb[...])

    return pl.pallas_call(
        body,
        out_shape=(
            jax.ShapeDtypeStruct((N, H), F32),
            jax.ShapeDtypeStruct((N, H), F32),
        ),
    )(h3, wea, web)


_BE = 3200


def _tc_mask(ge, l1, b1, l2, b2):
    def body(g, l1r, b1r, l2r, b2r, mref, eref):
        i = pl.program_id(0)
        ea = jnp.maximum(g[...], 0.0)
        u = jnp.maximum(_dot(ea, l1r[...]) + b1r[...], 0.0)
        t = _dot(u, l2r[...]) + b2r[...]
        m = 1.0 / (1.0 + jnp.exp(-t))
        mref[...] = m[:, 0]
        ent = -m * jnp.log(m + EPS) - (1.0 - m) * jnp.log(1.0 - m + EPS)
        tot = jnp.sum(ent)

        @pl.when(i == 0)
        def _():
            eref[0, 0] = tot

        @pl.when(i > 0)
        def _():
            eref[0, 0] = eref[0, 0] + tot

    grid = E // _BE
    return pl.pallas_call(
        body,
        grid=(grid,),
        in_specs=[
            pl.BlockSpec((_BE, H), lambda i: (i, 0)),
            pl.BlockSpec((H, 64), lambda i: (0, 0)),
            pl.BlockSpec((1, 64), lambda i: (0, 0)),
            pl.BlockSpec((64, 1), lambda i: (0, 0)),
            pl.BlockSpec((1, 1), lambda i: (0, 0)),
        ],
        out_specs=(
            pl.BlockSpec((_BE,), lambda i: (i,)),
            pl.BlockSpec((1, 1), lambda i: (0, 0)),
        ),
        out_shape=(
            jax.ShapeDtypeStruct((E,), F32),
            jax.ShapeDtypeStruct((1, 1), F32),
        ),
    )(ge, l1, b1, l2, b2)


def _tc_prep2(cntRm, eep):
    def body(cr, ee, out):
        out[...] = _dot(cr[0] + cr[1], ee[...])

    return pl.pallas_call(
        body, out_shape=jax.ShapeDtypeStruct((N, D), F32),
    )(cntRm, eep)


def _tc_layerm(b_p, cecm, g, m, s, dinv):
    def body(b, ce, gr, mr, sr, dv, out):
        bsum = b[0] + b[1] + ce[...]
        acc = _dot(bsum, mr[...]) * dv[...]
        out[...] = jnp.maximum(acc + _dot(gr[...], sr[...]), 0.0)

    return pl.pallas_call(
        body, out_shape=jax.ShapeDtypeStruct((N, D), F32),
    )(b_p, cecm, g, m, s, dinv)


def _tc_final(g3, batch2):
    def body(gr, br, out):
        bb = br[...]
        gi = lax.broadcasted_iota(jnp.int32, (1, G), 1)
        oh = (bb == gi).astype(F32)
        sums = lax.dot_general(oh, gr[...], (((0,), (0,)), ((), ())),
                               precision=PREC, preferred_element_type=F32)
        cnt = jnp.maximum(jnp.sum(oh, axis=0), 1.0).reshape(G, 1)
        out[...] = sums / cnt

    return pl.pallas_call(
        body, out_shape=jax.ShapeDtypeStruct((G, D), F32),
    )(g3, batch2)


def kernel(x, edge_index, edge_type, batch, prototype, edge_emb, W1, Ws1,
           W2, Ws2, W3, Ws3, Wedge, L1, b1, L2, b2, M1, S1, M2, S2, M3, S3):
    src = edge_index[0]
    dst = edge_index[1]
    eep = jnp.concatenate(
        [edge_emb, jnp.zeros((128 - (R + 1), D), F32)], axis=0)
    w1a = W1[:D]
    w1b = W1[D:]
    wea = Wedge[:H]
    web = Wedge[H:]

    cntR, cntG = _sc_counts(src, dst, edge_type, batch)
    a1 = _sc_seg_rows(x, src, dst)

    dinv, u1 = _tc_prep1(cntR, cntG, eep, prototype, w1a, w1b)
    h1 = _tc_layer1(a1, u1, x, w1a, Ws1, dinv)

    a2 = _sc_seg_rows(h1, src, dst)
    h2 = _tc_layer(a2, h1, W2, Ws2, dinv)

    a3 = _sc_seg_rows(h2, src, dst)
    h3 = _tc_layer(a3, h2, W3, Ws3, dinv)

    ha, hb = _tc_ab(h3, wea, web)
    ge = _sc_edge_feat(ha, hb, src, dst)
    edgemask, ent = _tc_mask(ge, L1, b1.reshape(1, 64), L2, b2.reshape(1, 1))

    cntRm = _sc_wcounts(dst, edge_type, edgemask)
    cecm = _tc_prep2(cntRm, eep)

    b1p = _sc_seg_rows(x, src, dst, mask=edgemask)
    g1 = _tc_layerm(b1p, cecm, x, M1, S1, dinv)

    b2p = _sc_seg_rows(g1, src, dst, mask=edgemask)
    g2 = _tc_layerm(b2p, cecm, g1, M2, S2, dinv)

    b3p = _sc_seg_rows(g2, src, dst, mask=edgemask)
    g3 = _tc_layerm(b3p, cecm, g2, M3, S3, dinv)

    emb = _tc_final(g3, batch.reshape(N, 1))
    extra_loss = ent.reshape(())
    return (emb, extra_loss, edgemask)


# speedup vs baseline: 3.6457x; 3.6457x over previous
import functools

import jax
import jax.numpy as jnp
from jax import lax
from jax.experimental import pallas as pl
from jax.experimental.pallas import tpu as pltpu
from jax.experimental.pallas import tpu_sc as plsc

N = 10000
E = 320000
D = 128
P = 64
H = 128
R = 101
G = 16

NC = 2
NS = 16
NW = NC * NS
EW = E // NW
K = 80
CHUNKS = EW // K
NP = 10240
RT = NP // NS
ZR = 128
NPR = NP * 128
NPG = NP * G
F32 = jnp.float32
EPS = 1e-15
PREC = lax.Precision.HIGHEST

_MESH = plsc.VectorSubcoreMesh(core_axis_name="c", subcore_axis_name="s")


def _dot(a, b):
    return jnp.dot(a, b, preferred_element_type=F32, precision=PREC)


def _zero_acc_rows(zbuf, acc, s):
    zv = jnp.zeros((16,), F32)

    def zrow(i, carry):
        for j in range(D // 16):
            zbuf[i, pl.ds(j * 16, 16)] = zv
        return carry

    lax.fori_loop(0, ZR, zrow, 0)
    for k in range(RT // ZR):
        pltpu.sync_copy(zbuf, acc.at[pl.ds(s * RT + k * ZR, ZR)])


def _copy_out_rows(zbuf, acc, out_hbm, c, s):
    for k in range(RT // ZR):
        sl = pl.ds(s * RT + k * ZR, ZR)
        pltpu.sync_copy(acc.at[sl], zbuf)
        pltpu.sync_copy(zbuf, out_hbm.at[c, sl])


def _sc_seg_rows(h, src, dst, mask=None):
    has_mask = mask is not None

    def body(*refs):
        if has_mask:
            (h_hbm, src_hbm, dst_hbm, m_hbm, out_hbm,
             sbuf, dbuf, mbuf, rows, zbuf, acc, sem) = refs
        else:
            (h_hbm, src_hbm, dst_hbm, out_hbm,
             sbuf, dbuf, rows, zbuf, acc, sem) = refs
        c = lax.axis_index("c")
        s = lax.axis_index("s")
        wid = c * NS + s
        _zero_acc_rows(zbuf, acc, s)
        plsc.subcore_barrier()

        def chunk(ci, carry):
            base = wid * EW + ci * K
            pltpu.sync_copy(src_hbm.at[pl.ds(base, K)], sbuf)
            pltpu.sync_copy(dst_hbm.at[pl.ds(base, K)], dbuf)
            pltpu.async_copy(h_hbm.at[sbuf], rows, sem).wait()
            if has_mask:
                pltpu.sync_copy(m_hbm.at[pl.ds(base, K)], mbuf)

                def mrow(g, cc):
                    mv = mbuf[pl.ds(g * 16, 16)]
                    for lane in range(16):
                        m = mv[lane]
                        i = g * 16 + lane
                        for j in range(D // 16):
                            sl = pl.ds(j * 16, 16)
                            rows[i, sl] = rows[i, sl] * m
                    return cc

                lax.fori_loop(0, K // 16, mrow, 0)
            pltpu.sync_copy(rows, acc.at[dbuf], add=True)
            return carry

        lax.fori_loop(0, CHUNKS, chunk, 0)
        plsc.subcore_barrier()
        _copy_out_rows(zbuf, acc, out_hbm, c, s)

    scratch = [
        pltpu.VMEM((K,), jnp.int32),
        pltpu.VMEM((K,), jnp.int32),
    ]
    if has_mask:
        scratch.append(pltpu.VMEM((K,), F32))
    scratch += [
        pltpu.VMEM((K, D), F32),
        pltpu.VMEM((ZR, D), F32),
        pltpu.VMEM_SHARED((NP, D), F32),
        pltpu.SemaphoreType.DMA,
    ]
    fn = pl.kernel(
        body,
        out_type=jax.ShapeDtypeStruct((NC, NP, D), F32),
        mesh=_MESH,
        scratch_types=scratch,
    )
    args = (h, src, dst) + ((mask,) if has_mask else ())
    return fn(*args)[:, :N]


def _sc_counts(src, dst, etype, batch):
    def body(src_hbm, dst_hbm, t_hbm, b_hbm, outR_hbm, outG_hbm,
             sbuf, dbuf, tbuf, gbuf, fR, fG, ones, zb, accR, accG, sem):
        c = lax.axis_index("c")
        s = lax.axis_index("s")
        wid = c * NS + s
        zv = jnp.zeros((16,), F32)
        ov = jnp.ones((16,), F32)

        def zfill(i, carry):
            zb[pl.ds(i * 16, 16)] = zv
            return carry

        lax.fori_loop(0, NP // 16, zfill, 0)
        for j in range(K // 16):
            ones[pl.ds(j * 16, 16)] = ov
        for k in range(8):
            pltpu.sync_copy(zb, accR.at[pl.ds(s * (8 * NP) + k * NP, NP)])
        pltpu.sync_copy(zb, accG.at[pl.ds(s * NP, NP)])
        plsc.subcore_barrier()

        def chunk(ci, carry):
            base = wid * EW + ci * K
            pltpu.sync_copy(src_hbm.at[pl.ds(base, K)], sbuf)
            pltpu.sync_copy(dst_hbm.at[pl.ds(base, K)], dbuf)
            pltpu.sync_copy(t_hbm.at[pl.ds(base, K)], tbuf)
            pltpu.async_copy(b_hbm.at[sbuf], gbuf, sem).wait()
            for g in range(K // 16):
                sl = pl.ds(g * 16, 16)
                d16 = dbuf[sl]
                fR[sl] = d16 * 128 + tbuf[sl]
                fG[sl] = d16 * G + gbuf[sl]
            pltpu.sync_copy(ones, accR.at[fR], add=True)
            pltpu.sync_copy(ones, accG.at[fG], add=True)
            return carry

        lax.fori_loop(0, CHUNKS, chunk, 0)
        plsc.subcore_barrier()
        for k in range(8):
            pltpu.sync_copy(accR.at[pl.ds(s * (8 * NP) + k * NP, NP)], zb)
            pltpu.sync_copy(
                zb, outR_hbm.at[pl.ds(c * NPR + s * (8 * NP) + k * NP, NP)])
        pltpu.sync_copy(accG.at[pl.ds(s * NP, NP)], zb)
        pltpu.sync_copy(zb, outG_hbm.at[pl.ds(c * NPG + s * NP, NP)])

    fn = pl.kernel(
        body,
        out_type=(
            jax.ShapeDtypeStruct((NC * NPR,), F32),
            jax.ShapeDtypeStruct((NC * NPG,), F32),
        ),
        mesh=_MESH,
        scratch_types=[
            pltpu.VMEM((K,), jnp.int32),
            pltpu.VMEM((K,), jnp.int32),
            pltpu.VMEM((K,), jnp.int32),
            pltpu.VMEM((K,), jnp.int32),
            pltpu.VMEM((K,), jnp.int32),
            pltpu.VMEM((K,), jnp.int32),
            pltpu.VMEM((K,), F32),
            pltpu.VMEM((NP,), F32),
            pltpu.VMEM_SHARED((NPR,), F32),
            pltpu.VMEM_SHARED((NPG,), F32),
            pltpu.SemaphoreType.DMA,
        ],
    )
    outR, outG = fn(src, dst, etype, batch)
    return (outR.reshape(NC, NP, 128)[:, :N],
            outG.reshape(NC, NP, G)[:, :N])


def _sc_wcounts(dst, etype, mask):
    def body(dst_hbm, t_hbm, m_hbm, outR_hbm,
             dbuf, tbuf, fR, wbuf, zb, accR):
        c = lax.axis_index("c")
        s = lax.axis_index("s")
        wid = c * NS + s
        zv = jnp.zeros((16,), F32)

        def zfill(i, carry):
            zb[pl.ds(i * 16, 16)] = zv
            return carry

        lax.fori_loop(0, NP // 16, zfill, 0)
        for k in range(8):
            pltpu.sync_copy(zb, accR.at[pl.ds(s * (8 * NP) + k * NP, NP)])
        plsc.subcore_barrier()

        def chunk(ci, carry):
            base = wid * EW + ci * K
            pltpu.sync_copy(dst_hbm.at[pl.ds(base, K)], dbuf)
            pltpu.sync_copy(t_hbm.at[pl.ds(base, K)], tbuf)
            pltpu.sync_copy(m_hbm.at[pl.ds(base, K)], wbuf)
            for g in range(K // 16):
                sl = pl.ds(g * 16, 16)
                fR[sl] = dbuf[sl] * 128 + tbuf[sl]
            pltpu.sync_copy(wbuf, accR.at[fR], add=True)
            return carry

        lax.fori_loop(0, CHUNKS, chunk, 0)
        plsc.subcore_barrier()
        for k in range(8):
            pltpu.sync_copy(accR.at[pl.ds(s * (8 * NP) + k * NP, NP)], zb)
            pltpu.sync_copy(
                zb, outR_hbm.at[pl.ds(c * NPR + s * (8 * NP) + k * NP, NP)])

    fn = pl.kernel(
        body,
        out_type=jax.ShapeDtypeStruct((NC * NPR,), F32),
        mesh=_MESH,
        scratch_types=[
            pltpu.VMEM((K,), jnp.int32),
            pltpu.VMEM((K,), jnp.int32),
            pltpu.VMEM((K,), jnp.int32),
            pltpu.VMEM((K,), F32),
            pltpu.VMEM((NP,), F32),
            pltpu.VMEM_SHARED((NPR,), F32),
        ],
    )
    return fn(dst, etype, mask).reshape(NC, NP, 128)[:, :N]


def _sc_edge_feat(ha, hb, src, dst):
    def body(ha_hbm, hb_hbm, src_hbm, dst_hbm, out_hbm,
             sbuf, dbuf, rows_a, rows_b, sema, semb):
        c = lax.axis_index("c")
        s = lax.axis_index("s")
        wid = c * NS + s

        def chunk(ci, carry):
            base = wid * EW + ci * K
            pltpu.sync_copy(src_hbm.at[pl.ds(base, K)], sbuf)
            pltpu.sync_copy(dst_hbm.at[pl.ds(base, K)], dbuf)
            da = pltpu.async_copy(ha_hbm.at[sbuf], rows_a, sema)
            db = pltpu.async_copy(hb_hbm.at[dbuf], rows_b, semb)
            da.wait()
            db.wait()

            def arow(i, cc):
                for j in range(H // 16):
                    sl = pl.ds(j * 16, 16)
                    rows_a[i, sl] = rows_a[i, sl] + rows_b[i, sl]
                return cc

            lax.fori_loop(0, K, arow, 0)
            pltpu.sync_copy(rows_a, out_hbm.at[pl.ds(base, K)])
            return carry

        lax.fori_loop(0, CHUNKS, chunk, 0)

    fn = pl.kernel(
        body,
        out_type=jax.ShapeDtypeStruct((E, H), F32),
        mesh=_MESH,
        scratch_types=[
            pltpu.VMEM((K,), jnp.int32),
            pltpu.VMEM((K,), jnp.int32),
            pltpu.VMEM((K, H), F32),
            pltpu.VMEM((K, H), F32),
            pltpu.SemaphoreType.DMA,
            pltpu.SemaphoreType.DMA,
        ],
    )
    return fn(ha, hb, src, dst)


_BN = 2000


def _nblk(shape):
    if len(shape) == 3:
        return pl.BlockSpec((shape[0], _BN, shape[2]), lambda i: (0, i, 0))
    return pl.BlockSpec(shape, lambda i: (i, 0))


def _wblk(shape):
    return pl.BlockSpec(shape, lambda i: (0, 0))


def _tc_prep1(cntR, cntG, eep, proto, w1a, w1b):
    def body(cr, cg, ee, pr, wa, wb, dinv, u1):
        crs = cr[0] + cr[1]
        cgs = cg[0] + cg[1]
        deg = jnp.maximum(jnp.sum(crs, axis=1, keepdims=True), 1.0)
        dinv[...] = 1.0 / deg
        u1[...] = _dot(_dot(crs, ee[...]), wa[...]) + \
            _dot(_dot(cgs, pr[...]), wb[...])

    return pl.pallas_call(
        body,
        grid=(N // _BN,),
        in_specs=[
            _nblk((2, _BN, 128)),
            _nblk((2, _BN, G)),
            _wblk((128, D)),
            _wblk((G, P)),
            _wblk((D, H)),
            _wblk((P, H)),
        ],
        out_specs=(_nblk((_BN, 1)), _nblk((_BN, H))),
        out_shape=(
            jax.ShapeDtypeStruct((N, 1), F32),
            jax.ShapeDtypeStruct((N, H), F32),
        ),
    )(cntR, cntG, eep, proto, w1a, w1b)


def _tc_layer1(a_p, u1, x, w1a, ws1, dinv):
    def body(a, u, xr, wa, ws, dv, out):
        asum = a[0] + a[1]
        acc = (_dot(asum, wa[...]) + u[...]) * dv[...]
        out[...] = jnp.maximum(acc + _dot(xr[...], ws[...]), 0.0)

    return pl.pallas_call(
        body,
        grid=(N // _BN,),
        in_specs=[
            _nblk((2, _BN, D)),
            _nblk((_BN, H)),
            _nblk((_BN, D)),
            _wblk((D, H)),
            _wblk((D, H)),
            _nblk((_BN, 1)),
        ],
        out_specs=_nblk((_BN, H)),
        out_shape=jax.ShapeDtypeStruct((N, H), F32),
    )(a_p, u1, x, w1a, ws1, dinv)


def _tc_layer(a_p, h, w, ws, dinv):
    def body(a, hr, wr, ws_r, dv, out):
        asum = a[0] + a[1]
        acc = _dot(asum, wr[...]) * dv[...]
        out[...] = jnp.maximum(acc + _dot(hr[...], ws_r[...]), 0.0)

    return pl.pallas_call(
        body,
        grid=(N // _BN,),
        in_specs=[
            _nblk((2, _BN, H)),
            _nblk((_BN, H)),
            _wblk((H, H)),
            _wblk((H, H)),
            _nblk((_BN, 1)),
        ],
        out_specs=_nblk((_BN, H)),
        out_shape=jax.ShapeDtypeStruct((N, H), F32),
    )(a_p, h, w, ws, dinv)


def _tc_ab(h3, wea, web):
    def body(hr, wa, wb, oa, ob):
        oa[...] = _dot(hr[...], wa[...])
        ob[...] = _dot(hr[...], wb[...])

    return pl.pallas_call(
        body,
        grid=(N // _BN,),
        in_specs=[_nblk((_BN, H)), _wblk((H, H)), _wblk((H, H))],
        out_specs=(_nblk((_BN, H)), _nblk((_BN, H))),
        out_shape=(
            jax.ShapeDtypeStruct((N, H), F32),
            jax.ShapeDtypeStruct((N, H), F32),
        ),
    )(h3, wea, web)


_BE = 3200


def _tc_mask(ge, l1, b1, l2, b2):
    def body(g, l1r, b1r, l2r, b2r, mref, eref):
        i = pl.program_id(0)
        ea = jnp.maximum(g[...], 0.0)
        u = jnp.maximum(_dot(ea, l1r[...]) + b1r[...], 0.0)
        t = _dot(u, l2r[...]) + b2r[...]
        m = 1.0 / (1.0 + jnp.exp(-t))
        mref[...] = m
        ent = -m * jnp.log(m + EPS) - (1.0 - m) * jnp.log(1.0 - m + EPS)
        tot = jnp.sum(ent, axis=(0, 1), keepdims=True)

        @pl.when(i == 0)
        def _():
            eref[...] = tot

        @pl.when(i > 0)
        def _():
            eref[...] = eref[...] + tot

    grid = E // _BE
    return pl.pallas_call(
        body,
        grid=(grid,),
        in_specs=[
            pl.BlockSpec((_BE, H), lambda i: (i, 0)),
            pl.BlockSpec((H, 64), lambda i: (0, 0)),
            pl.BlockSpec((1, 64), lambda i: (0, 0)),
            pl.BlockSpec((64, 1), lambda i: (0, 0)),
            pl.BlockSpec((1, 1), lambda i: (0, 0)),
        ],
        out_specs=(
            pl.BlockSpec((_BE, 1), lambda i: (i, 0)),
            pl.BlockSpec((1, 1), lambda i: (0, 0)),
        ),
        out_shape=(
            jax.ShapeDtypeStruct((E, 1), F32),
            jax.ShapeDtypeStruct((1, 1), F32),
        ),
    )(ge, l1, b1, l2, b2)


def _tc_prep2(cntRm, eep):
    def body(cr, ee, out):
        out[...] = _dot(cr[0] + cr[1], ee[...])

    return pl.pallas_call(
        body,
        grid=(N // _BN,),
        in_specs=[_nblk((2, _BN, 128)), _wblk((128, D))],
        out_specs=_nblk((_BN, D)),
        out_shape=jax.ShapeDtypeStruct((N, D), F32),
    )(cntRm, eep)


def _tc_layerm(b_p, cecm, g, m, s, dinv):
    def body(b, ce, gr, mr, sr, dv, out):
        bsum = b[0] + b[1] + ce[...]
        acc = _dot(bsum, mr[...]) * dv[...]
        out[...] = jnp.maximum(acc + _dot(gr[...], sr[...]), 0.0)

    return pl.pallas_call(
        body,
        grid=(N // _BN,),
        in_specs=[
            _nblk((2, _BN, D)),
            _nblk((_BN, D)),
            _nblk((_BN, D)),
            _wblk((D, D)),
            _wblk((D, D)),
            _nblk((_BN, 1)),
        ],
        out_specs=_nblk((_BN, D)),
        out_shape=jax.ShapeDtypeStruct((N, D), F32),
    )(b_p, cecm, g, m, s, dinv)


def _tc_final(g3, batch2):
    def body(gr, br, out):
        bb = br[...]
        gi = lax.broadcasted_iota(jnp.int32, (1, G), 1)
        oh = (bb == gi).astype(F32)
        sums = lax.dot_general(oh, gr[...], (((0,), (0,)), ((), ())),
                               precision=PREC, preferred_element_type=F32)
        cnt = jnp.maximum(jnp.sum(oh, axis=0), 1.0).reshape(G, 1)
        out[...] = sums / cnt

    return pl.pallas_call(
        body, out_shape=jax.ShapeDtypeStruct((G, D), F32),
    )(g3, batch2)


def kernel(x, edge_index, edge_type, batch, prototype, edge_emb, W1, Ws1,
           W2, Ws2, W3, Ws3, Wedge, L1, b1, L2, b2, M1, S1, M2, S2, M3, S3):
    src = edge_index[0]
    dst = edge_index[1]
    eep = jnp.concatenate(
        [edge_emb, jnp.zeros((128 - (R + 1), D), F32)], axis=0)
    w1a = W1[:D]
    w1b = W1[D:]
    wea = Wedge[:H]
    web = Wedge[H:]

    cntR, cntG = _sc_counts(src, dst, edge_type, batch)
    a1 = _sc_seg_rows(x, src, dst)

    dinv, u1 = _tc_prep1(cntR, cntG, eep, prototype, w1a, w1b)
    h1 = _tc_layer1(a1, u1, x, w1a, Ws1, dinv)

    a2 = _sc_seg_rows(h1, src, dst)
    h2 = _tc_layer(a2, h1, W2, Ws2, dinv)

    a3 = _sc_seg_rows(h2, src, dst)
    h3 = _tc_layer(a3, h2, W3, Ws3, dinv)

    ha, hb = _tc_ab(h3, wea, web)
    ge = _sc_edge_feat(ha, hb, src, dst)
    mask2, ent = _tc_mask(ge, L1, b1.reshape(1, 64), L2, b2.reshape(1, 1))
    edgemask = mask2.reshape(E)

    cntRm = _sc_wcounts(dst, edge_type, edgemask)
    cecm = _tc_prep2(cntRm, eep)

    b1p = _sc_seg_rows(x, src, dst, mask=edgemask)
    g1 = _tc_layerm(b1p, cecm, x, M1, S1, dinv)

    b2p = _sc_seg_rows(g1, src, dst, mask=edgemask)
    g2 = _tc_layerm(b2p, cecm, g1, M2, S2, dinv)

    b3p = _sc_seg_rows(g2, src, dst, mask=edgemask)
    g3 = _tc_layerm(b3p, cecm, g2, M3, S3, dinv)

    emb = _tc_final(g3, batch.reshape(N, 1))
    extra_loss = ent.reshape(())
    return (emb, extra_loss, edgemask)


# speedup vs baseline: 4.5102x; 1.2371x over previous
import functools

import jax
import jax.numpy as jnp
from jax import lax
from jax.experimental import pallas as pl
from jax.experimental.pallas import tpu as pltpu
from jax.experimental.pallas import tpu_sc as plsc

N = 10000
E = 320000
D = 128
P = 64
H = 128
R = 101
G = 16

NC = 2
NS = 16
NW = NC * NS
EW = E // NW
K = 80
CHUNKS = EW // K
NBUF = 4
NP = 10240
RT = NP // NS
ZR = 32
NPR = NP * 128
NPG = NP * G
F32 = jnp.float32
EPS = 1e-15
PREC = lax.Precision.HIGHEST

_MESH = plsc.VectorSubcoreMesh(core_axis_name="c", subcore_axis_name="s")


def _dot(a, b):
    return jnp.dot(a, b, preferred_element_type=F32, precision=PREC)


def _zero_acc_rows(zbuf, acc, s):
    zv = jnp.zeros((16,), F32)

    def zrow(i, carry):
        for j in range(D // 16):
            zbuf[i, pl.ds(j * 16, 16)] = zv
        return carry

    lax.fori_loop(0, ZR, zrow, 0)
    for k in range(RT // ZR):
        pltpu.sync_copy(zbuf, acc.at[pl.ds(s * RT + k * ZR, ZR)])


def _copy_out_rows(zbuf, acc, out_hbm, c, s):
    for k in range(RT // ZR):
        sl = pl.ds(s * RT + k * ZR, ZR)
        pltpu.sync_copy(acc.at[sl], zbuf)
        pltpu.sync_copy(zbuf, out_hbm.at[c, sl])


def _sc_seg_rows(h, src, dst, mask=None):
    has_mask = mask is not None

    def body(*refs):
        if has_mask:
            (h_hbm, src_hbm, dst_hbm, m_hbm, out_hbm,
             sbufs, dbufs, mbufs, rowss, zbuf, acc, gsems) = refs
        else:
            (h_hbm, src_hbm, dst_hbm, out_hbm,
             sbufs, dbufs, rowss, zbuf, acc, gsems) = refs
        c = lax.axis_index("c")
        s = lax.axis_index("s")
        wid = c * NS + s
        _zero_acc_rows(zbuf, acc, s)
        plsc.subcore_barrier()

        def fire(b, ci):
            base = wid * EW + ci * K
            pltpu.sync_copy(src_hbm.at[pl.ds(base, K)], sbufs[b])
            pltpu.async_copy(h_hbm.at[sbufs[b]], rowss[b], gsems[b])

        def process(b, ci):
            base = wid * EW + ci * K
            pltpu.make_async_copy(
                h_hbm.at[sbufs[b]], rowss[b], gsems[b]).wait()
            pltpu.sync_copy(dst_hbm.at[pl.ds(base, K)], dbufs[b])
            if has_mask:
                pltpu.sync_copy(m_hbm.at[pl.ds(base, K)], mbufs[b])

                def mrow(gg, cc):
                    mv = mbufs[b][pl.ds(gg * 16, 16)]
                    for lane in range(16):
                        m = mv[lane]
                        i = gg * 16 + lane
                        for j in range(D // 16):
                            sl = pl.ds(j * 16, 16)
                            rowss[b][i, sl] = rowss[b][i, sl] * m
                    return cc

                lax.fori_loop(0, K // 16, mrow, 0)
            pltpu.sync_copy(rowss[b], acc.at[dbufs[b]], add=True)

        for b in range(NBUF):
            fire(b, b)

        def grp(g, carry):
            for b in range(NBUF):
                ci = g * NBUF + b
                process(b, ci)
                nci = ci + NBUF

                @pl.when(nci < CHUNKS)
                def _(b=b, nci=nci):
                    fire(b, nci)
            return carry

        lax.fori_loop(0, CHUNKS // NBUF, grp, 0)
        for b in range(CHUNKS % NBUF):
            process(b, (CHUNKS // NBUF) * NBUF + b)
        plsc.subcore_barrier()
        _copy_out_rows(zbuf, acc, out_hbm, c, s)

    scratch = [
        [pltpu.VMEM((K,), jnp.int32)] * NBUF,
        [pltpu.VMEM((K,), jnp.int32)] * NBUF,
    ]
    if has_mask:
        scratch.append([pltpu.VMEM((K,), F32)] * NBUF)
    scratch += [
        [pltpu.VMEM((K, D), F32)] * NBUF,
        pltpu.VMEM((ZR, D), F32),
        pltpu.VMEM_SHARED((NP, D), F32),
        [pltpu.SemaphoreType.DMA] * NBUF,
    ]
    fn = pl.kernel(
        body,
        out_type=jax.ShapeDtypeStruct((NC, NP, D), F32),
        mesh=_MESH,
        scratch_types=scratch,
    )
    args = (h, src, dst) + ((mask,) if has_mask else ())
    return fn(*args)[:, :N]


def _sc_counts(src, dst, etype, batch):
    def body(src_hbm, dst_hbm, t_hbm, b_hbm, outR_hbm, outG_hbm,
             sbuf, dbuf, tbuf, gbuf, fR, fG, ones, zb, accR, accG, sem):
        c = lax.axis_index("c")
        s = lax.axis_index("s")
        wid = c * NS + s
        zv = jnp.zeros((16,), F32)
        ov = jnp.ones((16,), F32)

        def zfill(i, carry):
            zb[pl.ds(i * 16, 16)] = zv
            return carry

        lax.fori_loop(0, NP // 16, zfill, 0)
        for j in range(K // 16):
            ones[pl.ds(j * 16, 16)] = ov
        for k in range(8):
            pltpu.sync_copy(zb, accR.at[pl.ds(s * (8 * NP) + k * NP, NP)])
        pltpu.sync_copy(zb, accG.at[pl.ds(s * NP, NP)])
        plsc.subcore_barrier()

        def chunk(ci, carry):
            base = wid * EW + ci * K
            pltpu.sync_copy(src_hbm.at[pl.ds(base, K)], sbuf)
            pltpu.sync_copy(dst_hbm.at[pl.ds(base, K)], dbuf)
            pltpu.sync_copy(t_hbm.at[pl.ds(base, K)], tbuf)
            pltpu.async_copy(b_hbm.at[sbuf], gbuf, sem).wait()
            for g in range(K // 16):
                sl = pl.ds(g * 16, 16)
                d16 = dbuf[sl]
                fR[sl] = d16 * 128 + tbuf[sl]
                fG[sl] = d16 * G + gbuf[sl]
            pltpu.sync_copy(ones, accR.at[fR], add=True)
            pltpu.sync_copy(ones, accG.at[fG], add=True)
            return carry

        lax.fori_loop(0, CHUNKS, chunk, 0)
        plsc.subcore_barrier()
        for k in range(8):
            pltpu.sync_copy(accR.at[pl.ds(s * (8 * NP) + k * NP, NP)], zb)
            pltpu.sync_copy(
                zb, outR_hbm.at[pl.ds(c * NPR + s * (8 * NP) + k * NP, NP)])
        pltpu.sync_copy(accG.at[pl.ds(s * NP, NP)], zb)
        pltpu.sync_copy(zb, outG_hbm.at[pl.ds(c * NPG + s * NP, NP)])

    fn = pl.kernel(
        body,
        out_type=(
            jax.ShapeDtypeStruct((NC * NPR,), F32),
            jax.ShapeDtypeStruct((NC * NPG,), F32),
        ),
        mesh=_MESH,
        scratch_types=[
            pltpu.VMEM((K,), jnp.int32),
            pltpu.VMEM((K,), jnp.int32),
            pltpu.VMEM((K,), jnp.int32),
            pltpu.VMEM((K,), jnp.int32),
            pltpu.VMEM((K,), jnp.int32),
            pltpu.VMEM((K,), jnp.int32),
            pltpu.VMEM((K,), F32),
            pltpu.VMEM((NP,), F32),
            pltpu.VMEM_SHARED((NPR,), F32),
            pltpu.VMEM_SHARED((NPG,), F32),
            pltpu.SemaphoreType.DMA,
        ],
    )
    outR, outG = fn(src, dst, etype, batch)
    return (outR.reshape(NC, NP, 128)[:, :N],
            outG.reshape(NC, NP, G)[:, :N])


def _sc_wcounts(dst, etype, mask):
    def body(dst_hbm, t_hbm, m_hbm, outR_hbm,
             dbuf, tbuf, fR, wbuf, zb, accR):
        c = lax.axis_index("c")
        s = lax.axis_index("s")
        wid = c * NS + s
        zv = jnp.zeros((16,), F32)

        def zfill(i, carry):
            zb[pl.ds(i * 16, 16)] = zv
            return carry

        lax.fori_loop(0, NP // 16, zfill, 0)
        for k in range(8):
            pltpu.sync_copy(zb, accR.at[pl.ds(s * (8 * NP) + k * NP, NP)])
        plsc.subcore_barrier()

        def chunk(ci, carry):
            base = wid * EW + ci * K
            pltpu.sync_copy(dst_hbm.at[pl.ds(base, K)], dbuf)
            pltpu.sync_copy(t_hbm.at[pl.ds(base, K)], tbuf)
            pltpu.sync_copy(m_hbm.at[pl.ds(base, K)], wbuf)
            for g in range(K // 16):
                sl = pl.ds(g * 16, 16)
                fR[sl] = dbuf[sl] * 128 + tbuf[sl]
            pltpu.sync_copy(wbuf, accR.at[fR], add=True)
            return carry

        lax.fori_loop(0, CHUNKS, chunk, 0)
        plsc.subcore_barrier()
        for k in range(8):
            pltpu.sync_copy(accR.at[pl.ds(s * (8 * NP) + k * NP, NP)], zb)
            pltpu.sync_copy(
                zb, outR_hbm.at[pl.ds(c * NPR + s * (8 * NP) + k * NP, NP)])

    fn = pl.kernel(
        body,
        out_type=jax.ShapeDtypeStruct((NC * NPR,), F32),
        mesh=_MESH,
        scratch_types=[
            pltpu.VMEM((K,), jnp.int32),
            pltpu.VMEM((K,), jnp.int32),
            pltpu.VMEM((K,), jnp.int32),
            pltpu.VMEM((K,), F32),
            pltpu.VMEM((NP,), F32),
            pltpu.VMEM_SHARED((NPR,), F32),
        ],
    )
    return fn(dst, etype, mask).reshape(NC, NP, 128)[:, :N]


def _sc_edge_feat(ha, hb, src, dst):
    def body(ha_hbm, hb_hbm, src_hbm, dst_hbm, out_hbm,
             sbuf, dbuf, rows_a, rows_b, sema, semb):
        c = lax.axis_index("c")
        s = lax.axis_index("s")
        wid = c * NS + s

        def chunk(ci, carry):
            base = wid * EW + ci * K
            pltpu.sync_copy(src_hbm.at[pl.ds(base, K)], sbuf)
            pltpu.sync_copy(dst_hbm.at[pl.ds(base, K)], dbuf)
            da = pltpu.async_copy(ha_hbm.at[sbuf], rows_a, sema)
            db = pltpu.async_copy(hb_hbm.at[dbuf], rows_b, semb)
            da.wait()
            db.wait()

            def arow(i, cc):
                for j in range(H // 16):
                    sl = pl.ds(j * 16, 16)
                    rows_a[i, sl] = rows_a[i, sl] + rows_b[i, sl]
                return cc

            lax.fori_loop(0, K, arow, 0)
            pltpu.sync_copy(rows_a, out_hbm.at[pl.ds(base, K)])
            return carry

        lax.fori_loop(0, CHUNKS, chunk, 0)

    fn = pl.kernel(
        body,
        out_type=jax.ShapeDtypeStruct((E, H), F32),
        mesh=_MESH,
        scratch_types=[
            pltpu.VMEM((K,), jnp.int32),
            pltpu.VMEM((K,), jnp.int32),
            pltpu.VMEM((K, H), F32),
            pltpu.VMEM((K, H), F32),
            pltpu.SemaphoreType.DMA,
            pltpu.SemaphoreType.DMA,
        ],
    )
    return fn(ha, hb, src, dst)


_BN = 2000


def _nblk(shape):
    if len(shape) == 3:
        return pl.BlockSpec((shape[0], _BN, shape[2]), lambda i: (0, i, 0))
    return pl.BlockSpec(shape, lambda i: (i, 0))


def _wblk(shape):
    return pl.BlockSpec(shape, lambda i: (0, 0))


def _tc_prep1(cntR, cntG, eep, proto, w1a, w1b):
    def body(cr, cg, ee, pr, wa, wb, dinv, u1):
        crs = cr[0] + cr[1]
        cgs = cg[0] + cg[1]
        deg = jnp.maximum(jnp.sum(crs, axis=1, keepdims=True), 1.0)
        dinv[...] = 1.0 / deg
        u1[...] = _dot(_dot(crs, ee[...]), wa[...]) + \
            _dot(_dot(cgs, pr[...]), wb[...])

    return pl.pallas_call(
        body,
        grid=(N // _BN,),
        in_specs=[
            _nblk((2, _BN, 128)),
            _nblk((2, _BN, G)),
            _wblk((128, D)),
            _wblk((G, P)),
            _wblk((D, H)),
            _wblk((P, H)),
        ],
        out_specs=(_nblk((_BN, 1)), _nblk((_BN, H))),
        out_shape=(
            jax.ShapeDtypeStruct((N, 1), F32),
            jax.ShapeDtypeStruct((N, H), F32),
        ),
    )(cntR, cntG, eep, proto, w1a, w1b)


def _tc_layer1(a_p, u1, x, w1a, ws1, dinv):
    def body(a, u, xr, wa, ws, dv, out):
        asum = a[0] + a[1]
        acc = (_dot(asum, wa[...]) + u[...]) * dv[...]
        out[...] = jnp.maximum(acc + _dot(xr[...], ws[...]), 0.0)

    return pl.pallas_call(
        body,
        grid=(N // _BN,),
        in_specs=[
            _nblk((2, _BN, D)),
            _nblk((_BN, H)),
            _nblk((_BN, D)),
            _wblk((D, H)),
            _wblk((D, H)),
            _nblk((_BN, 1)),
        ],
        out_specs=_nblk((_BN, H)),
        out_shape=jax.ShapeDtypeStruct((N, H), F32),
    )(a_p, u1, x, w1a, ws1, dinv)


def _tc_layer(a_p, h, w, ws, dinv):
    def body(a, hr, wr, ws_r, dv, out):
        asum = a[0] + a[1]
        acc = _dot(asum, wr[...]) * dv[...]
        out[...] = jnp.maximum(acc + _dot(hr[...], ws_r[...]), 0.0)

    return pl.pallas_call(
        body,
        grid=(N // _BN,),
        in_specs=[
            _nblk((2, _BN, H)),
            _nblk((_BN, H)),
            _wblk((H, H)),
            _wblk((H, H)),
            _nblk((_BN, 1)),
        ],
        out_specs=_nblk((_BN, H)),
        out_shape=jax.ShapeDtypeStruct((N, H), F32),
    )(a_p, h, w, ws, dinv)


def _tc_ab(h3, wea, web):
    def body(hr, wa, wb, oa, ob):
        oa[...] = _dot(hr[...], wa[...])
        ob[...] = _dot(hr[...], wb[...])

    return pl.pallas_call(
        body,
        grid=(N // _BN,),
        in_specs=[_nblk((_BN, H)), _wblk((H, H)), _wblk((H, H))],
        out_specs=(_nblk((_BN, H)), _nblk((_BN, H))),
        out_shape=(
            jax.ShapeDtypeStruct((N, H), F32),
            jax.ShapeDtypeStruct((N, H), F32),
        ),
    )(h3, wea, web)


_BE = 3200


def _tc_mask(ge, l1, b1, l2, b2):
    def body(g, l1r, b1r, l2r, b2r, mref, eref):
        i = pl.program_id(0)
        ea = jnp.maximum(g[...], 0.0)
        u = jnp.maximum(_dot(ea, l1r[...]) + b1r[...], 0.0)
        t = _dot(u, l2r[...]) + b2r[...]
        m = 1.0 / (1.0 + jnp.exp(-t))
        mref[...] = m
        ent = -m * jnp.log(m + EPS) - (1.0 - m) * jnp.log(1.0 - m + EPS)
        tot = jnp.sum(ent, axis=(0, 1), keepdims=True)

        @pl.when(i == 0)
        def _():
            eref[...] = tot

        @pl.when(i > 0)
        def _():
            eref[...] = eref[...] + tot

    grid = E // _BE
    return pl.pallas_call(
        body,
        grid=(grid,),
        in_specs=[
            pl.BlockSpec((_BE, H), lambda i: (i, 0)),
            pl.BlockSpec((H, 64), lambda i: (0, 0)),
            pl.BlockSpec((1, 64), lambda i: (0, 0)),
            pl.BlockSpec((64, 1), lambda i: (0, 0)),
            pl.BlockSpec((1, 1), lambda i: (0, 0)),
        ],
        out_specs=(
            pl.BlockSpec((_BE, 1), lambda i: (i, 0)),
            pl.BlockSpec((1, 1), lambda i: (0, 0)),
        ),
        out_shape=(
            jax.ShapeDtypeStruct((E, 1), F32),
            jax.ShapeDtypeStruct((1, 1), F32),
        ),
    )(ge, l1, b1, l2, b2)


def _tc_prep2(cntRm, eep):
    def body(cr, ee, out):
        out[...] = _dot(cr[0] + cr[1], ee[...])

    return pl.pallas_call(
        body,
        grid=(N // _BN,),
        in_specs=[_nblk((2, _BN, 128)), _wblk((128, D))],
        out_specs=_nblk((_BN, D)),
        out_shape=jax.ShapeDtypeStruct((N, D), F32),
    )(cntRm, eep)


def _tc_layerm(b_p, cecm, g, m, s, dinv):
    def body(b, ce, gr, mr, sr, dv, out):
        bsum = b[0] + b[1] + ce[...]
        acc = _dot(bsum, mr[...]) * dv[...]
        out[...] = jnp.maximum(acc + _dot(gr[...], sr[...]), 0.0)

    return pl.pallas_call(
        body,
        grid=(N // _BN,),
        in_specs=[
            _nblk((2, _BN, D)),
            _nblk((_BN, D)),
            _nblk((_BN, D)),
            _wblk((D, D)),
            _wblk((D, D)),
            _nblk((_BN, 1)),
        ],
        out_specs=_nblk((_BN, D)),
        out_shape=jax.ShapeDtypeStruct((N, D), F32),
    )(b_p, cecm, g, m, s, dinv)


def _tc_final(g3, batch2):
    def body(gr, br, out):
        bb = br[...]
        gi = lax.broadcasted_iota(jnp.int32, (1, G), 1)
        oh = (bb == gi).astype(F32)
        sums = lax.dot_general(oh, gr[...], (((0,), (0,)), ((), ())),
                               precision=PREC, preferred_element_type=F32)
        cnt = jnp.maximum(jnp.sum(oh, axis=0), 1.0).reshape(G, 1)
        out[...] = sums / cnt

    return pl.pallas_call(
        body, out_shape=jax.ShapeDtypeStruct((G, D), F32),
    )(g3, batch2)


def kernel(x, edge_index, edge_type, batch, prototype, edge_emb, W1, Ws1,
           W2, Ws2, W3, Ws3, Wedge, L1, b1, L2, b2, M1, S1, M2, S2, M3, S3):
    src = edge_index[0]
    dst = edge_index[1]
    eep = jnp.concatenate(
        [edge_emb, jnp.zeros((128 - (R + 1), D), F32)], axis=0)
    w1a = W1[:D]
    w1b = W1[D:]
    wea = Wedge[:H]
    web = Wedge[H:]

    cntR, cntG = _sc_counts(src, dst, edge_type, batch)
    a1 = _sc_seg_rows(x, src, dst)

    dinv, u1 = _tc_prep1(cntR, cntG, eep, prototype, w1a, w1b)
    h1 = _tc_layer1(a1, u1, x, w1a, Ws1, dinv)

    a2 = _sc_seg_rows(h1, src, dst)
    h2 = _tc_layer(a2, h1, W2, Ws2, dinv)

    a3 = _sc_seg_rows(h2, src, dst)
    h3 = _tc_layer(a3, h2, W3, Ws3, dinv)

    ha, hb = _tc_ab(h3, wea, web)
    ge = _sc_edge_feat(ha, hb, src, dst)
    mask2, ent = _tc_mask(ge, L1, b1.reshape(1, 64), L2, b2.reshape(1, 1))
    edgemask = mask2.reshape(E)

    cntRm = _sc_wcounts(dst, edge_type, edgemask)
    cecm = _tc_prep2(cntRm, eep)

    b1p = _sc_seg_rows(x, src, dst, mask=edgemask)
    g1 = _tc_layerm(b1p, cecm, x, M1, S1, dinv)

    b2p = _sc_seg_rows(g1, src, dst, mask=edgemask)
    g2 = _tc_layerm(b2p, cecm, g1, M2, S2, dinv)

    b3p = _sc_seg_rows(g2, src, dst, mask=edgemask)
    g3 = _tc_layerm(b3p, cecm, g2, M3, S3, dinv)

    emb = _tc_final(g3, batch.reshape(N, 1))
    extra_loss = ent.reshape(())
    return (emb, extra_loss, edgemask)


# speedup vs baseline: 4.5372x; 1.0060x over previous
import functools

import jax
import jax.numpy as jnp
from jax import lax
from jax.experimental import pallas as pl
from jax.experimental.pallas import tpu as pltpu
from jax.experimental.pallas import tpu_sc as plsc

N = 10000
E = 320000
D = 128
P = 64
H = 128
R = 101
G = 16

NC = 2
NS = 16
NW = NC * NS
EW = E // NW
K = 80
CHUNKS = EW // K
NBUF = 4
NP = 10240
RT = NP // NS
ZR = 32
NPR = NP * 128
NPG = NP * G
F32 = jnp.float32
EPS = 1e-15
PREC = lax.Precision.HIGHEST

_MESH = plsc.VectorSubcoreMesh(core_axis_name="c", subcore_axis_name="s")


def _dot(a, b):
    return jnp.dot(a, b, preferred_element_type=F32, precision=PREC)


def _zero_acc_rows(zbuf, acc, s):
    zv = jnp.zeros((16,), F32)

    def zrow(i, carry):
        for j in range(D // 16):
            zbuf[i, pl.ds(j * 16, 16)] = zv
        return carry

    lax.fori_loop(0, ZR, zrow, 0)
    for k in range(RT // ZR):
        pltpu.sync_copy(zbuf, acc.at[pl.ds(s * RT + k * ZR, ZR)])


def _copy_out_rows(zbuf, acc, out_hbm, c, s):
    sl = pl.ds(s * RT, RT)
    pltpu.sync_copy(acc.at[sl], out_hbm.at[c, sl])


def _sc_seg_rows(h, src, dst, mask=None):
    has_mask = mask is not None

    def body(*refs):
        if has_mask:
            (h_hbm, src_hbm, dst_hbm, m_hbm, out_hbm,
             sbufs, dbufs, mbufs, rowss, zbuf, acc, gsems) = refs
        else:
            (h_hbm, src_hbm, dst_hbm, out_hbm,
             sbufs, dbufs, rowss, zbuf, acc, gsems) = refs
        c = lax.axis_index("c")
        s = lax.axis_index("s")
        wid = c * NS + s
        _zero_acc_rows(zbuf, acc, s)
        plsc.subcore_barrier()

        def fire(b, ci):
            base = wid * EW + ci * K
            pltpu.sync_copy(src_hbm.at[pl.ds(base, K)], sbufs[b])
            pltpu.async_copy(h_hbm.at[sbufs[b]], rowss[b], gsems[b])

        def process(b, ci):
            base = wid * EW + ci * K
            pltpu.make_async_copy(
                h_hbm.at[sbufs[b]], rowss[b], gsems[b]).wait()
            pltpu.sync_copy(dst_hbm.at[pl.ds(base, K)], dbufs[b])
            if has_mask:
                pltpu.sync_copy(m_hbm.at[pl.ds(base, K)], mbufs[b])

                def mrow(gg, cc):
                    mv = mbufs[b][pl.ds(gg * 16, 16)]
                    for lane in range(16):
                        m = mv[lane]
                        i = gg * 16 + lane
                        for j in range(D // 16):
                            sl = pl.ds(j * 16, 16)
                            rowss[b][i, sl] = rowss[b][i, sl] * m
                    return cc

                lax.fori_loop(0, K // 16, mrow, 0)
            pltpu.sync_copy(rowss[b], acc.at[dbufs[b]], add=True)

        for b in range(NBUF):
            fire(b, b)

        def grp(g, carry):
            for b in range(NBUF):
                ci = g * NBUF + b
                process(b, ci)
                nci = ci + NBUF

                @pl.when(nci < CHUNKS)
                def _(b=b, nci=nci):
                    fire(b, nci)
            return carry

        lax.fori_loop(0, CHUNKS // NBUF, grp, 0)
        for b in range(CHUNKS % NBUF):
            process(b, (CHUNKS // NBUF) * NBUF + b)
        plsc.subcore_barrier()
        _copy_out_rows(zbuf, acc, out_hbm, c, s)

    scratch = [
        [pltpu.VMEM((K,), jnp.int32)] * NBUF,
        [pltpu.VMEM((K,), jnp.int32)] * NBUF,
    ]
    if has_mask:
        scratch.append([pltpu.VMEM((K,), F32)] * NBUF)
    scratch += [
        [pltpu.VMEM((K, D), F32)] * NBUF,
        pltpu.VMEM((ZR, D), F32),
        pltpu.VMEM_SHARED((NP, D), F32),
        [pltpu.SemaphoreType.DMA] * NBUF,
    ]
    fn = pl.kernel(
        body,
        out_type=jax.ShapeDtypeStruct((NC, NP, D), F32),
        mesh=_MESH,
        scratch_types=scratch,
    )
    args = (h, src, dst) + ((mask,) if has_mask else ())
    return fn(*args)[:, :N]


def _sc_counts(src, dst, etype, batch):
    def body(src_hbm, dst_hbm, t_hbm, b_hbm, outR_hbm, outG_hbm,
             sbuf, dbuf, tbuf, gbuf, fR, fG, ones, zb, accR, accG, sem):
        c = lax.axis_index("c")
        s = lax.axis_index("s")
        wid = c * NS + s
        zv = jnp.zeros((16,), F32)
        ov = jnp.ones((16,), F32)

        def zfill(i, carry):
            zb[pl.ds(i * 16, 16)] = zv
            return carry

        lax.fori_loop(0, NP // 16, zfill, 0)
        for j in range(K // 16):
            ones[pl.ds(j * 16, 16)] = ov
        for k in range(8):
            pltpu.sync_copy(zb, accR.at[pl.ds(s * (8 * NP) + k * NP, NP)])
        pltpu.sync_copy(zb, accG.at[pl.ds(s * NP, NP)])
        plsc.subcore_barrier()

        def chunk(ci, carry):
            base = wid * EW + ci * K
            pltpu.sync_copy(src_hbm.at[pl.ds(base, K)], sbuf)
            pltpu.sync_copy(dst_hbm.at[pl.ds(base, K)], dbuf)
            pltpu.sync_copy(t_hbm.at[pl.ds(base, K)], tbuf)
            pltpu.async_copy(b_hbm.at[sbuf], gbuf, sem).wait()
            for g in range(K // 16):
                sl = pl.ds(g * 16, 16)
                d16 = dbuf[sl]
                fR[sl] = d16 * 128 + tbuf[sl]
                fG[sl] = d16 * G + gbuf[sl]
            pltpu.sync_copy(ones, accR.at[fR], add=True)
            pltpu.sync_copy(ones, accG.at[fG], add=True)
            return carry

        lax.fori_loop(0, CHUNKS, chunk, 0)
        plsc.subcore_barrier()
        pltpu.sync_copy(accR.at[pl.ds(s * (8 * NP), 8 * NP)],
                        outR_hbm.at[pl.ds(c * NPR + s * (8 * NP), 8 * NP)])
        pltpu.sync_copy(accG.at[pl.ds(s * NP, NP)],
                        outG_hbm.at[pl.ds(c * NPG + s * NP, NP)])

    fn = pl.kernel(
        body,
        out_type=(
            jax.ShapeDtypeStruct((NC * NPR,), F32),
            jax.ShapeDtypeStruct((NC * NPG,), F32),
        ),
        mesh=_MESH,
        scratch_types=[
            pltpu.VMEM((K,), jnp.int32),
            pltpu.VMEM((K,), jnp.int32),
            pltpu.VMEM((K,), jnp.int32),
            pltpu.VMEM((K,), jnp.int32),
            pltpu.VMEM((K,), jnp.int32),
            pltpu.VMEM((K,), jnp.int32),
            pltpu.VMEM((K,), F32),
            pltpu.VMEM((NP,), F32),
            pltpu.VMEM_SHARED((NPR,), F32),
            pltpu.VMEM_SHARED((NPG,), F32),
            pltpu.SemaphoreType.DMA,
        ],
    )
    outR, outG = fn(src, dst, etype, batch)
    return (outR.reshape(NC, NP, 128)[:, :N],
            outG.reshape(NC, NP, G)[:, :N])


def _sc_wcounts(dst, etype, mask):
    def body(dst_hbm, t_hbm, m_hbm, outR_hbm,
             dbuf, tbuf, fR, wbuf, zb, accR):
        c = lax.axis_index("c")
        s = lax.axis_index("s")
        wid = c * NS + s
        zv = jnp.zeros((16,), F32)

        def zfill(i, carry):
            zb[pl.ds(i * 16, 16)] = zv
            return carry

        lax.fori_loop(0, NP // 16, zfill, 0)
        for k in range(8):
            pltpu.sync_copy(zb, accR.at[pl.ds(s * (8 * NP) + k * NP, NP)])
        plsc.subcore_barrier()

        def chunk(ci, carry):
            base = wid * EW + ci * K
            pltpu.sync_copy(dst_hbm.at[pl.ds(base, K)], dbuf)
            pltpu.sync_copy(t_hbm.at[pl.ds(base, K)], tbuf)
            pltpu.sync_copy(m_hbm.at[pl.ds(base, K)], wbuf)
            for g in range(K // 16):
                sl = pl.ds(g * 16, 16)
                fR[sl] = dbuf[sl] * 128 + tbuf[sl]
            pltpu.sync_copy(wbuf, accR.at[fR], add=True)
            return carry

        lax.fori_loop(0, CHUNKS, chunk, 0)
        plsc.subcore_barrier()
        pltpu.sync_copy(accR.at[pl.ds(s * (8 * NP), 8 * NP)],
                        outR_hbm.at[pl.ds(c * NPR + s * (8 * NP), 8 * NP)])

    fn = pl.kernel(
        body,
        out_type=jax.ShapeDtypeStruct((NC * NPR,), F32),
        mesh=_MESH,
        scratch_types=[
            pltpu.VMEM((K,), jnp.int32),
            pltpu.VMEM((K,), jnp.int32),
            pltpu.VMEM((K,), jnp.int32),
            pltpu.VMEM((K,), F32),
            pltpu.VMEM((NP,), F32),
            pltpu.VMEM_SHARED((NPR,), F32),
        ],
    )
    return fn(dst, etype, mask).reshape(NC, NP, 128)[:, :N]


def _sc_edge_feat(ha, hb, src, dst):
    def body(ha_hbm, hb_hbm, src_hbm, dst_hbm, out_hbm,
             sbuf, dbuf, rows_a, rows_b, sema, semb):
        c = lax.axis_index("c")
        s = lax.axis_index("s")
        wid = c * NS + s

        def chunk(ci, carry):
            base = wid * EW + ci * K
            pltpu.sync_copy(src_hbm.at[pl.ds(base, K)], sbuf)
            pltpu.sync_copy(dst_hbm.at[pl.ds(base, K)], dbuf)
            da = pltpu.async_copy(ha_hbm.at[sbuf], rows_a, sema)
            db = pltpu.async_copy(hb_hbm.at[dbuf], rows_b, semb)
            da.wait()
            db.wait()

            def arow(i, cc):
                for j in range(H // 16):
                    sl = pl.ds(j * 16, 16)
                    rows_a[i, sl] = rows_a[i, sl] + rows_b[i, sl]
                return cc

            lax.fori_loop(0, K, arow, 0)
            pltpu.sync_copy(rows_a, out_hbm.at[pl.ds(base, K)])
            return carry

        lax.fori_loop(0, CHUNKS, chunk, 0)

    fn = pl.kernel(
        body,
        out_type=jax.ShapeDtypeStruct((E, H), F32),
        mesh=_MESH,
        scratch_types=[
            pltpu.VMEM((K,), jnp.int32),
            pltpu.VMEM((K,), jnp.int32),
            pltpu.VMEM((K, H), F32),
            pltpu.VMEM((K, H), F32),
            pltpu.SemaphoreType.DMA,
            pltpu.SemaphoreType.DMA,
        ],
    )
    return fn(ha, hb, src, dst)


_BN = 2000


def _nblk(shape):
    if len(shape) == 3:
        return pl.BlockSpec((shape[0], _BN, shape[2]), lambda i: (0, i, 0))
    return pl.BlockSpec(shape, lambda i: (i, 0))


def _wblk(shape):
    return pl.BlockSpec(shape, lambda i: (0, 0))


def _tc_prep1(cntR, cntG, eep, proto, w1a, w1b):
    def body(cr, cg, ee, pr, wa, wb, dinv, u1):
        crs = cr[0] + cr[1]
        cgs = cg[0] + cg[1]
        deg = jnp.maximum(jnp.sum(crs, axis=1, keepdims=True), 1.0)
        dinv[...] = 1.0 / deg
        u1[...] = _dot(_dot(crs, ee[...]), wa[...]) + \
            _dot(_dot(cgs, pr[...]), wb[...])

    return pl.pallas_call(
        body,
        grid=(N // _BN,),
        in_specs=[
            _nblk((2, _BN, 128)),
            _nblk((2, _BN, G)),
            _wblk((128, D)),
            _wblk((G, P)),
            _wblk((D, H)),
            _wblk((P, H)),
        ],
        out_specs=(_nblk((_BN, 1)), _nblk((_BN, H))),
        out_shape=(
            jax.ShapeDtypeStruct((N, 1), F32),
            jax.ShapeDtypeStruct((N, H), F32),
        ),
    )(cntR, cntG, eep, proto, w1a, w1b)


def _tc_layer1(a_p, u1, x, w1a, ws1, dinv):
    def body(a, u, xr, wa, ws, dv, out):
        asum = a[0] + a[1]
        acc = (_dot(asum, wa[...]) + u[...]) * dv[...]
        out[...] = jnp.maximum(acc + _dot(xr[...], ws[...]), 0.0)

    return pl.pallas_call(
        body,
        grid=(N // _BN,),
        in_specs=[
            _nblk((2, _BN, D)),
            _nblk((_BN, H)),
            _nblk((_BN, D)),
            _wblk((D, H)),
            _wblk((D, H)),
            _nblk((_BN, 1)),
        ],
        out_specs=_nblk((_BN, H)),
        out_shape=jax.ShapeDtypeStruct((N, H), F32),
    )(a_p, u1, x, w1a, ws1, dinv)


def _tc_layer(a_p, h, w, ws, dinv):
    def body(a, hr, wr, ws_r, dv, out):
        asum = a[0] + a[1]
        acc = _dot(asum, wr[...]) * dv[...]
        out[...] = jnp.maximum(acc + _dot(hr[...], ws_r[...]), 0.0)

    return pl.pallas_call(
        body,
        grid=(N // _BN,),
        in_specs=[
            _nblk((2, _BN, H)),
            _nblk((_BN, H)),
            _wblk((H, H)),
            _wblk((H, H)),
            _nblk((_BN, 1)),
        ],
        out_specs=_nblk((_BN, H)),
        out_shape=jax.ShapeDtypeStruct((N, H), F32),
    )(a_p, h, w, ws, dinv)


def _tc_ab(h3, wea, web):
    def body(hr, wa, wb, oa, ob):
        oa[...] = _dot(hr[...], wa[...])
        ob[...] = _dot(hr[...], wb[...])

    return pl.pallas_call(
        body,
        grid=(N // _BN,),
        in_specs=[_nblk((_BN, H)), _wblk((H, H)), _wblk((H, H))],
        out_specs=(_nblk((_BN, H)), _nblk((_BN, H))),
        out_shape=(
            jax.ShapeDtypeStruct((N, H), F32),
            jax.ShapeDtypeStruct((N, H), F32),
        ),
    )(h3, wea, web)


_BE = 3200


def _tc_mask(ge, l1, b1, l2, b2):
    def body(g, l1r, b1r, l2r, b2r, mref, eref):
        i = pl.program_id(0)
        ea = jnp.maximum(g[...], 0.0)
        u = jnp.maximum(_dot(ea, l1r[...]) + b1r[...], 0.0)
        t = _dot(u, l2r[...]) + b2r[...]
        m = 1.0 / (1.0 + jnp.exp(-t))
        mref[...] = m
        ent = -m * jnp.log(m + EPS) - (1.0 - m) * jnp.log(1.0 - m + EPS)
        tot = jnp.sum(ent, axis=(0, 1), keepdims=True)

        @pl.when(i == 0)
        def _():
            eref[...] = tot

        @pl.when(i > 0)
        def _():
            eref[...] = eref[...] + tot

    grid = E // _BE
    return pl.pallas_call(
        body,
        grid=(grid,),
        in_specs=[
            pl.BlockSpec((_BE, H), lambda i: (i, 0)),
            pl.BlockSpec((H, 64), lambda i: (0, 0)),
            pl.BlockSpec((1, 64), lambda i: (0, 0)),
            pl.BlockSpec((64, 1), lambda i: (0, 0)),
            pl.BlockSpec((1, 1), lambda i: (0, 0)),
        ],
        out_specs=(
            pl.BlockSpec((_BE, 1), lambda i: (i, 0)),
            pl.BlockSpec((1, 1), lambda i: (0, 0)),
        ),
        out_shape=(
            jax.ShapeDtypeStruct((E, 1), F32),
            jax.ShapeDtypeStruct((1, 1), F32),
        ),
    )(ge, l1, b1, l2, b2)


def _tc_prep2(cntRm, eep):
    def body(cr, ee, out):
        out[...] = _dot(cr[0] + cr[1], ee[...])

    return pl.pallas_call(
        body,
        grid=(N // _BN,),
        in_specs=[_nblk((2, _BN, 128)), _wblk((128, D))],
        out_specs=_nblk((_BN, D)),
        out_shape=jax.ShapeDtypeStruct((N, D), F32),
    )(cntRm, eep)


def _tc_layerm(b_p, cecm, g, m, s, dinv):
    def body(b, ce, gr, mr, sr, dv, out):
        bsum = b[0] + b[1] + ce[...]
        acc = _dot(bsum, mr[...]) * dv[...]
        out[...] = jnp.maximum(acc + _dot(gr[...], sr[...]), 0.0)

    return pl.pallas_call(
        body,
        grid=(N // _BN,),
        in_specs=[
            _nblk((2, _BN, D)),
            _nblk((_BN, D)),
            _nblk((_BN, D)),
            _wblk((D, D)),
            _wblk((D, D)),
            _nblk((_BN, 1)),
        ],
        out_specs=_nblk((_BN, D)),
        out_shape=jax.ShapeDtypeStruct((N, D), F32),
    )(b_p, cecm, g, m, s, dinv)


def _tc_final(g3, batch2):
    def body(gr, br, out):
        bb = br[...]
        gi = lax.broadcasted_iota(jnp.int32, (1, G), 1)
        oh = (bb == gi).astype(F32)
        sums = lax.dot_general(oh, gr[...], (((0,), (0,)), ((), ())),
                               precision=PREC, preferred_element_type=F32)
        cnt = jnp.maximum(jnp.sum(oh, axis=0), 1.0).reshape(G, 1)
        out[...] = sums / cnt

    return pl.pallas_call(
        body, out_shape=jax.ShapeDtypeStruct((G, D), F32),
    )(g3, batch2)


def kernel(x, edge_index, edge_type, batch, prototype, edge_emb, W1, Ws1,
           W2, Ws2, W3, Ws3, Wedge, L1, b1, L2, b2, M1, S1, M2, S2, M3, S3):
    src = edge_index[0]
    dst = edge_index[1]
    eep = jnp.concatenate(
        [edge_emb, jnp.zeros((128 - (R + 1), D), F32)], axis=0)
    w1a = W1[:D]
    w1b = W1[D:]
    wea = Wedge[:H]
    web = Wedge[H:]

    cntR, cntG = _sc_counts(src, dst, edge_type, batch)
    a1 = _sc_seg_rows(x, src, dst)

    dinv, u1 = _tc_prep1(cntR, cntG, eep, prototype, w1a, w1b)
    h1 = _tc_layer1(a1, u1, x, w1a, Ws1, dinv)

    a2 = _sc_seg_rows(h1, src, dst)
    h2 = _tc_layer(a2, h1, W2, Ws2, dinv)

    a3 = _sc_seg_rows(h2, src, dst)
    h3 = _tc_layer(a3, h2, W3, Ws3, dinv)

    ha, hb = _tc_ab(h3, wea, web)
    ge = _sc_edge_feat(ha, hb, src, dst)
    mask2, ent = _tc_mask(ge, L1, b1.reshape(1, 64), L2, b2.reshape(1, 1))
    edgemask = mask2.reshape(E)

    cntRm = _sc_wcounts(dst, edge_type, edgemask)
    cecm = _tc_prep2(cntRm, eep)

    b1p = _sc_seg_rows(x, src, dst, mask=edgemask)
    g1 = _tc_layerm(b1p, cecm, x, M1, S1, dinv)

    b2p = _sc_seg_rows(g1, src, dst, mask=edgemask)
    g2 = _tc_layerm(b2p, cecm, g1, M2, S2, dinv)

    b3p = _sc_seg_rows(g2, src, dst, mask=edgemask)
    g3 = _tc_layerm(b3p, cecm, g2, M3, S3, dinv)

    emb = _tc_final(g3, batch.reshape(N, 1))
    extra_loss = ent.reshape(())
    return (emb, extra_loss, edgemask)


# speedup vs baseline: 5.2216x; 1.1508x over previous
import functools

import jax
import jax.numpy as jnp
from jax import lax
from jax.experimental import pallas as pl
from jax.experimental.pallas import tpu as pltpu
from jax.experimental.pallas import tpu_sc as plsc

N = 10000
E = 320000
D = 128
P = 64
H = 128
R = 101
G = 16

NC = 2
NS = 16
NW = NC * NS
EW = E // NW
K = 80
CHUNKS = EW // K
NBUF = 4
NP = 10240
RT = NP // NS
ZR = 32
NPR = NP * 128
NPG = NP * G
F32 = jnp.float32
EPS = 1e-15
PREC = lax.Precision.HIGHEST

_MESH = plsc.VectorSubcoreMesh(core_axis_name="c", subcore_axis_name="s")


def _dot(a, b):
    return jnp.dot(a, b, preferred_element_type=F32, precision=PREC)


def _zero_acc_rows(zbuf, acc, s, zsem):
    zv = jnp.zeros((16,), F32)

    def zrow(i, carry):
        for j in range(D // 16):
            zbuf[i, pl.ds(j * 16, 16)] = zv
        return carry

    lax.fori_loop(0, ZR, zrow, 0)
    copies = [
        pltpu.async_copy(zbuf, acc.at[pl.ds(s * RT + k * ZR, ZR)], zsem)
        for k in range(RT // ZR)
    ]
    for cp in copies:
        cp.wait()


def _copy_out_rows(zbuf, acc, out_hbm, c, s):
    sl = pl.ds(s * RT, RT)
    pltpu.sync_copy(acc.at[sl], out_hbm.at[c, sl])


def _sc_seg_rows(h, src, dst, mask=None):
    has_mask = mask is not None

    def body(*refs):
        if has_mask:
            (h_hbm, src_hbm, dst_hbm, m_hbm, out_hbm,
             sbufs, dbufs, mbufs, rowss, zbuf, acc, gsems, zsem) = refs
        else:
            (h_hbm, src_hbm, dst_hbm, out_hbm,
             sbufs, dbufs, rowss, zbuf, acc, gsems, zsem) = refs
        c = lax.axis_index("c")
        s = lax.axis_index("s")
        wid = c * NS + s

        def fire(b, ci):
            base = wid * EW + ci * K
            pltpu.sync_copy(src_hbm.at[pl.ds(base, K)], sbufs[b])
            pltpu.async_copy(h_hbm.at[sbufs[b]], rowss[b], gsems[b])

        for b in range(NBUF):
            fire(b, b)
        _zero_acc_rows(zbuf, acc, s, zsem)
        plsc.subcore_barrier()

        def process(b, ci):
            base = wid * EW + ci * K
            pltpu.make_async_copy(
                h_hbm.at[sbufs[b]], rowss[b], gsems[b]).wait()
            pltpu.sync_copy(dst_hbm.at[pl.ds(base, K)], dbufs[b])
            if has_mask:
                pltpu.sync_copy(m_hbm.at[pl.ds(base, K)], mbufs[b])

                def mrow(gg, cc):
                    mv = mbufs[b][pl.ds(gg * 16, 16)]
                    for lane in range(16):
                        m = mv[lane]
                        i = gg * 16 + lane
                        for j in range(D // 16):
                            sl = pl.ds(j * 16, 16)
                            rowss[b][i, sl] = rowss[b][i, sl] * m
                    return cc

                lax.fori_loop(0, K // 16, mrow, 0)
            pltpu.sync_copy(rowss[b], acc.at[dbufs[b]], add=True)

        def grp(g, carry):
            for b in range(NBUF):
                ci = g * NBUF + b
                process(b, ci)
                nci = ci + NBUF

                @pl.when(nci < CHUNKS)
                def _(b=b, nci=nci):
                    fire(b, nci)
            return carry

        lax.fori_loop(0, CHUNKS // NBUF, grp, 0)
        for b in range(CHUNKS % NBUF):
            process(b, (CHUNKS // NBUF) * NBUF + b)
        plsc.subcore_barrier()
        _copy_out_rows(zbuf, acc, out_hbm, c, s)

    scratch = [
        [pltpu.VMEM((K,), jnp.int32)] * NBUF,
        [pltpu.VMEM((K,), jnp.int32)] * NBUF,
    ]
    if has_mask:
        scratch.append([pltpu.VMEM((K,), F32)] * NBUF)
    scratch += [
        [pltpu.VMEM((K, D), F32)] * NBUF,
        pltpu.VMEM((ZR, D), F32),
        pltpu.VMEM_SHARED((NP, D), F32),
        [pltpu.SemaphoreType.DMA] * NBUF,
        pltpu.SemaphoreType.DMA,
    ]
    fn = pl.kernel(
        body,
        out_type=jax.ShapeDtypeStruct((NC, NP, D), F32),
        mesh=_MESH,
        scratch_types=scratch,
    )
    args = (h, src, dst) + ((mask,) if has_mask else ())
    return fn(*args)[:, :N]


def _sc_counts(src, dst, etype, batch):
    def body(src_hbm, dst_hbm, t_hbm, b_hbm, outR_hbm, outG_hbm,
             sbuf, dbuf, tbuf, gbuf, fR, fG, ones, zb, accR, accG,
             sem, semd, semt):
        c = lax.axis_index("c")
        s = lax.axis_index("s")
        wid = c * NS + s
        zv = jnp.zeros((16,), F32)
        ov = jnp.ones((16,), F32)

        def zfill(i, carry):
            zb[pl.ds(i * 16, 16)] = zv
            return carry

        lax.fori_loop(0, NP // 16, zfill, 0)
        for j in range(K // 16):
            ones[pl.ds(j * 16, 16)] = ov
        zcs = [
            pltpu.async_copy(
                zb, accR.at[pl.ds(s * (8 * NP) + k * NP, NP)], semd)
            for k in range(8)
        ]
        zcs.append(pltpu.async_copy(zb, accG.at[pl.ds(s * NP, NP)], semt))
        for cp in zcs:
            cp.wait()
        plsc.subcore_barrier()

        def chunk(ci, carry):
            base = wid * EW + ci * K
            cs = pltpu.async_copy(src_hbm.at[pl.ds(base, K)], sbuf, sem)
            cd = pltpu.async_copy(dst_hbm.at[pl.ds(base, K)], dbuf, semd)
            ct = pltpu.async_copy(t_hbm.at[pl.ds(base, K)], tbuf, semt)
            cs.wait()
            cg = pltpu.async_copy(b_hbm.at[sbuf], gbuf, sem)
            cd.wait()
            ct.wait()
            for g in range(K // 16):
                sl = pl.ds(g * 16, 16)
                fR[sl] = dbuf[sl] * 128 + tbuf[sl]
            cg.wait()
            for g in range(K // 16):
                sl = pl.ds(g * 16, 16)
                fG[sl] = dbuf[sl] * G + gbuf[sl]
            pltpu.sync_copy(ones, accR.at[fR], add=True)
            pltpu.sync_copy(ones, accG.at[fG], add=True)
            return carry

        lax.fori_loop(0, CHUNKS, chunk, 0)
        plsc.subcore_barrier()
        pltpu.sync_copy(accR.at[pl.ds(s * (8 * NP), 8 * NP)],
                        outR_hbm.at[pl.ds(c * NPR + s * (8 * NP), 8 * NP)])
        pltpu.sync_copy(accG.at[pl.ds(s * NP, NP)],
                        outG_hbm.at[pl.ds(c * NPG + s * NP, NP)])

    fn = pl.kernel(
        body,
        out_type=(
            jax.ShapeDtypeStruct((NC * NPR,), F32),
            jax.ShapeDtypeStruct((NC * NPG,), F32),
        ),
        mesh=_MESH,
        scratch_types=[
            pltpu.VMEM((K,), jnp.int32),
            pltpu.VMEM((K,), jnp.int32),
            pltpu.VMEM((K,), jnp.int32),
            pltpu.VMEM((K,), jnp.int32),
            pltpu.VMEM((K,), jnp.int32),
            pltpu.VMEM((K,), jnp.int32),
            pltpu.VMEM((K,), F32),
            pltpu.VMEM((NP,), F32),
            pltpu.VMEM_SHARED((NPR,), F32),
            pltpu.VMEM_SHARED((NPG,), F32),
            pltpu.SemaphoreType.DMA,
            pltpu.SemaphoreType.DMA,
            pltpu.SemaphoreType.DMA,
        ],
    )
    outR, outG = fn(src, dst, etype, batch)
    return (outR.reshape(NC, NP, 128)[:, :N],
            outG.reshape(NC, NP, G)[:, :N])


def _sc_wcounts(dst, etype, mask):
    def body(dst_hbm, t_hbm, m_hbm, outR_hbm,
             dbuf, tbuf, fR, wbuf, zb, accR, semd, semt, semm):
        c = lax.axis_index("c")
        s = lax.axis_index("s")
        wid = c * NS + s
        zv = jnp.zeros((16,), F32)

        def zfill(i, carry):
            zb[pl.ds(i * 16, 16)] = zv
            return carry

        lax.fori_loop(0, NP // 16, zfill, 0)
        for k in range(8):
            pltpu.sync_copy(zb, accR.at[pl.ds(s * (8 * NP) + k * NP, NP)])
        plsc.subcore_barrier()

        def chunk(ci, carry):
            base = wid * EW + ci * K
            cd = pltpu.async_copy(dst_hbm.at[pl.ds(base, K)], dbuf, semd)
            ct = pltpu.async_copy(t_hbm.at[pl.ds(base, K)], tbuf, semt)
            cm = pltpu.async_copy(m_hbm.at[pl.ds(base, K)], wbuf, semm)
            cd.wait()
            ct.wait()
            for g in range(K // 16):
                sl = pl.ds(g * 16, 16)
                fR[sl] = dbuf[sl] * 128 + tbuf[sl]
            cm.wait()
            pltpu.sync_copy(wbuf, accR.at[fR], add=True)
            return carry

        lax.fori_loop(0, CHUNKS, chunk, 0)
        plsc.subcore_barrier()
        pltpu.sync_copy(accR.at[pl.ds(s * (8 * NP), 8 * NP)],
                        outR_hbm.at[pl.ds(c * NPR + s * (8 * NP), 8 * NP)])

    fn = pl.kernel(
        body,
        out_type=jax.ShapeDtypeStruct((NC * NPR,), F32),
        mesh=_MESH,
        scratch_types=[
            pltpu.VMEM((K,), jnp.int32),
            pltpu.VMEM((K,), jnp.int32),
            pltpu.VMEM((K,), jnp.int32),
            pltpu.VMEM((K,), F32),
            pltpu.VMEM((NP,), F32),
            pltpu.VMEM_SHARED((NPR,), F32),
            pltpu.SemaphoreType.DMA,
            pltpu.SemaphoreType.DMA,
            pltpu.SemaphoreType.DMA,
        ],
    )
    return fn(dst, etype, mask).reshape(NC, NP, 128)[:, :N]


def _sc_edge_feat(ha, hb, src, dst):
    NB2 = 3

    def body(ha_hbm, hb_hbm, src_hbm, dst_hbm, out_hbm,
             sbufs, dbufs, rows_a, rows_b, rows_o, semas, sembs, semos):
        c = lax.axis_index("c")
        s = lax.axis_index("s")
        wid = c * NS + s

        def fire(b, ci):
            base = wid * EW + ci * K
            pltpu.sync_copy(src_hbm.at[pl.ds(base, K)], sbufs[b])
            pltpu.sync_copy(dst_hbm.at[pl.ds(base, K)], dbufs[b])
            pltpu.async_copy(ha_hbm.at[sbufs[b]], rows_a[b], semas[b])
            pltpu.async_copy(hb_hbm.at[dbufs[b]], rows_b[b], sembs[b])

        def process(b, ci, drain):
            base = wid * EW + ci * K
            pltpu.make_async_copy(
                ha_hbm.at[sbufs[b]], rows_a[b], semas[b]).wait()
            pltpu.make_async_copy(
                hb_hbm.at[dbufs[b]], rows_b[b], sembs[b]).wait()
            if drain:
                pltpu.make_async_copy(
                    rows_o[b], out_hbm.at[pl.ds(0, K)], semos[b]).wait()

            def arow(i, cc):
                for j in range(H // 16):
                    sl = pl.ds(j * 16, 16)
                    rows_o[b][i, sl] = rows_a[b][i, sl] + rows_b[b][i, sl]
                return cc

            lax.fori_loop(0, K, arow, 0)
            pltpu.async_copy(rows_o[b], out_hbm.at[pl.ds(base, K)], semos[b])

        for b in range(NB2):
            fire(b, b)

        def grp(g, carry):
            for b in range(NB2):
                ci = g * NB2 + b
                process(b, ci, True)
                nci = ci + NB2

                @pl.when(nci < CHUNKS)
                def _(b=b, nci=nci):
                    fire(b, nci)
            return carry

        for b in range(NB2):
            process(b, b, False)
            fire(b, b + NB2)
        lax.fori_loop(1, CHUNKS // NB2, grp, 0)
        for b in range(CHUNKS % NB2):
            process(b, (CHUNKS // NB2) * NB2 + b, True)
        for b in range(NB2):
            pltpu.make_async_copy(
                rows_o[b], out_hbm.at[pl.ds(0, K)], semos[b]).wait()

    fn = pl.kernel(
        body,
        out_type=jax.ShapeDtypeStruct((E, H), F32),
        mesh=_MESH,
        scratch_types=[
            [pltpu.VMEM((K,), jnp.int32)] * NB2,
            [pltpu.VMEM((K,), jnp.int32)] * NB2,
            [pltpu.VMEM((K, H), F32)] * NB2,
            [pltpu.VMEM((K, H), F32)] * NB2,
            [pltpu.VMEM((K, H), F32)] * NB2,
            [pltpu.SemaphoreType.DMA] * NB2,
            [pltpu.SemaphoreType.DMA] * NB2,
            [pltpu.SemaphoreType.DMA] * NB2,
        ],
    )
    return fn(ha, hb, src, dst)


_BN = 2000


def _nblk(shape):
    if len(shape) == 3:
        return pl.BlockSpec((shape[0], _BN, shape[2]), lambda i: (0, i, 0))
    return pl.BlockSpec(shape, lambda i: (i, 0))


def _wblk(shape):
    return pl.BlockSpec(shape, lambda i: (0, 0))


def _tc_prep1(cntR, cntG, eep, proto, w1a, w1b):
    def body(cr, cg, ee, pr, wa, wb, dinv, u1):
        crs = cr[0] + cr[1]
        cgs = cg[0] + cg[1]
        deg = jnp.maximum(jnp.sum(crs, axis=1, keepdims=True), 1.0)
        dinv[...] = 1.0 / deg
        u1[...] = _dot(_dot(crs, ee[...]), wa[...]) + \
            _dot(_dot(cgs, pr[...]), wb[...])

    return pl.pallas_call(
        body,
        grid=(N // _BN,),
        in_specs=[
            _nblk((2, _BN, 128)),
            _nblk((2, _BN, G)),
            _wblk((128, D)),
            _wblk((G, P)),
            _wblk((D, H)),
            _wblk((P, H)),
        ],
        out_specs=(_nblk((_BN, 1)), _nblk((_BN, H))),
        out_shape=(
            jax.ShapeDtypeStruct((N, 1), F32),
            jax.ShapeDtypeStruct((N, H), F32),
        ),
    )(cntR, cntG, eep, proto, w1a, w1b)


def _tc_layer1(a_p, u1, x, w1a, ws1, dinv):
    def body(a, u, xr, wa, ws, dv, out):
        asum = a[0] + a[1]
        acc = (_dot(asum, wa[...]) + u[...]) * dv[...]
        out[...] = jnp.maximum(acc + _dot(xr[...], ws[...]), 0.0)

    return pl.pallas_call(
        body,
        grid=(N // _BN,),
        in_specs=[
            _nblk((2, _BN, D)),
            _nblk((_BN, H)),
            _nblk((_BN, D)),
            _wblk((D, H)),
            _wblk((D, H)),
            _nblk((_BN, 1)),
        ],
        out_specs=_nblk((_BN, H)),
        out_shape=jax.ShapeDtypeStruct((N, H), F32),
    )(a_p, u1, x, w1a, ws1, dinv)


def _tc_layer(a_p, h, w, ws, dinv):
    def body(a, hr, wr, ws_r, dv, out):
        asum = a[0] + a[1]
        acc = _dot(asum, wr[...]) * dv[...]
        out[...] = jnp.maximum(acc + _dot(hr[...], ws_r[...]), 0.0)

    return pl.pallas_call(
        body,
        grid=(N // _BN,),
        in_specs=[
            _nblk((2, _BN, H)),
            _nblk((_BN, H)),
            _wblk((H, H)),
            _wblk((H, H)),
            _nblk((_BN, 1)),
        ],
        out_specs=_nblk((_BN, H)),
        out_shape=jax.ShapeDtypeStruct((N, H), F32),
    )(a_p, h, w, ws, dinv)


def _tc_ab(h3, wea, web):
    def body(hr, wa, wb, oa, ob):
        oa[...] = _dot(hr[...], wa[...])
        ob[...] = _dot(hr[...], wb[...])

    return pl.pallas_call(
        body,
        grid=(N // _BN,),
        in_specs=[_nblk((_BN, H)), _wblk((H, H)), _wblk((H, H))],
        out_specs=(_nblk((_BN, H)), _nblk((_BN, H))),
        out_shape=(
            jax.ShapeDtypeStruct((N, H), F32),
            jax.ShapeDtypeStruct((N, H), F32),
        ),
    )(h3, wea, web)


_BE = 3200


def _tc_mask(ge, l1, b1, l2, b2):
    def body(g, l1r, b1r, l2r, b2r, mref, eref):
        i = pl.program_id(0)
        ea = jnp.maximum(g[...], 0.0)
        u = jnp.maximum(_dot(ea, l1r[...]) + b1r[...], 0.0)
        t = _dot(u, l2r[...]) + b2r[...]
        m = 1.0 / (1.0 + jnp.exp(-t))
        mref[...] = m
        ent = -m * jnp.log(m + EPS) - (1.0 - m) * jnp.log(1.0 - m + EPS)
        tot = jnp.sum(ent, axis=(0, 1), keepdims=True)

        @pl.when(i == 0)
        def _():
            eref[...] = tot

        @pl.when(i > 0)
        def _():
            eref[...] = eref[...] + tot

    grid = E // _BE
    return pl.pallas_call(
        body,
        grid=(grid,),
        in_specs=[
            pl.BlockSpec((_BE, H), lambda i: (i, 0)),
            pl.BlockSpec((H, 64), lambda i: (0, 0)),
            pl.BlockSpec((1, 64), lambda i: (0, 0)),
            pl.BlockSpec((64, 1), lambda i: (0, 0)),
            pl.BlockSpec((1, 1), lambda i: (0, 0)),
        ],
        out_specs=(
            pl.BlockSpec((_BE, 1), lambda i: (i, 0)),
            pl.BlockSpec((1, 1), lambda i: (0, 0)),
        ),
        out_shape=(
            jax.ShapeDtypeStruct((E, 1), F32),
            jax.ShapeDtypeStruct((1, 1), F32),
        ),
    )(ge, l1, b1, l2, b2)


def _tc_prep2(cntRm, eep):
    def body(cr, ee, out):
        out[...] = _dot(cr[0] + cr[1], ee[...])

    return pl.pallas_call(
        body,
        grid=(N // _BN,),
        in_specs=[_nblk((2, _BN, 128)), _wblk((128, D))],
        out_specs=_nblk((_BN, D)),
        out_shape=jax.ShapeDtypeStruct((N, D), F32),
    )(cntRm, eep)


def _tc_layerm(b_p, cecm, g, m, s, dinv):
    def body(b, ce, gr, mr, sr, dv, out):
        bsum = b[0] + b[1] + ce[...]
        acc = _dot(bsum, mr[...]) * dv[...]
        out[...] = jnp.maximum(acc + _dot(gr[...], sr[...]), 0.0)

    return pl.pallas_call(
        body,
        grid=(N // _BN,),
        in_specs=[
            _nblk((2, _BN, D)),
            _nblk((_BN, D)),
            _nblk((_BN, D)),
            _wblk((D, D)),
            _wblk((D, D)),
            _nblk((_BN, 1)),
        ],
        out_specs=_nblk((_BN, D)),
        out_shape=jax.ShapeDtypeStruct((N, D), F32),
    )(b_p, cecm, g, m, s, dinv)


def _tc_final(g3, batch2):
    def body(gr, br, out):
        bb = br[...]
        gi = lax.broadcasted_iota(jnp.int32, (1, G), 1)
        oh = (bb == gi).astype(F32)
        sums = lax.dot_general(oh, gr[...], (((0,), (0,)), ((), ())),
                               precision=PREC, preferred_element_type=F32)
        cnt = jnp.maximum(jnp.sum(oh, axis=0), 1.0).reshape(G, 1)
        out[...] = sums / cnt

    return pl.pallas_call(
        body, out_shape=jax.ShapeDtypeStruct((G, D), F32),
    )(g3, batch2)


def kernel(x, edge_index, edge_type, batch, prototype, edge_emb, W1, Ws1,
           W2, Ws2, W3, Ws3, Wedge, L1, b1, L2, b2, M1, S1, M2, S2, M3, S3):
    src = edge_index[0]
    dst = edge_index[1]
    eep = jnp.concatenate(
        [edge_emb, jnp.zeros((128 - (R + 1), D), F32)], axis=0)
    w1a = W1[:D]
    w1b = W1[D:]
    wea = Wedge[:H]
    web = Wedge[H:]

    cntR, cntG = _sc_counts(src, dst, edge_type, batch)
    a1 = _sc_seg_rows(x, src, dst)

    dinv, u1 = _tc_prep1(cntR, cntG, eep, prototype, w1a, w1b)
    h1 = _tc_layer1(a1, u1, x, w1a, Ws1, dinv)

    a2 = _sc_seg_rows(h1, src, dst)
    h2 = _tc_layer(a2, h1, W2, Ws2, dinv)

    a3 = _sc_seg_rows(h2, src, dst)
    h3 = _tc_layer(a3, h2, W3, Ws3, dinv)

    ha, hb = _tc_ab(h3, wea, web)
    ge = _sc_edge_feat(ha, hb, src, dst)
    mask2, ent = _tc_mask(ge, L1, b1.reshape(1, 64), L2, b2.reshape(1, 1))
    edgemask = mask2.reshape(E)

    cntRm = _sc_wcounts(dst, edge_type, edgemask)
    cecm = _tc_prep2(cntRm, eep)

    b1p = _sc_seg_rows(x, src, dst, mask=edgemask)
    g1 = _tc_layerm(b1p, cecm, x, M1, S1, dinv)

    b2p = _sc_seg_rows(g1, src, dst, mask=edgemask)
    g2 = _tc_layerm(b2p, cecm, g1, M2, S2, dinv)

    b3p = _sc_seg_rows(g2, src, dst, mask=edgemask)
    g3 = _tc_layerm(b3p, cecm, g2, M3, S3, dinv)

    emb = _tc_final(g3, batch.reshape(N, 1))
    extra_loss = ent.reshape(())
    return (emb, extra_loss, edgemask)
